# baseline (speedup 1.0000x reference)
import functools

import jax
import jax.numpy as jnp
from jax import lax
from jax.experimental import pallas as pl
from jax.experimental.pallas import tpu as pltpu
from jax.experimental.pallas import tpu_sc as plsc

O = 10000
T = 160000
H = 128
DOUT = 32
NCAPS = 8
NHID = 16
DREP = NCAPS * NHID
D1_OUT = 2 * H + DOUT
NLAYER = 2
ROUTIT = 3

TILE = 2000
NT = T // TILE

NC, NS = 2, 16
NW = NC * NS
CH = 128
NCHUNK = T // CH
HEXT = H + NHID
CHS = 128
NCHUNKS = T // CHS
SCHN = 624 // CHS
SREM = 624 - SCHN * CHS
F32 = jnp.float32


def _leaky(x):
    return jnp.where(x >= 0, x, 0.01 * x)


def _gmat():
    r = lax.broadcasted_iota(jnp.int32, (DREP, NCAPS), 0) // NHID
    c = lax.broadcasted_iota(jnp.int32, (DREP, NCAPS), 1)
    return (r == c).astype(F32)


def _emat():
    r = lax.broadcasted_iota(jnp.int32, (NCAPS, DREP), 0)
    c = lax.broadcasted_iota(jnp.int32, (NCAPS, DREP), 1) // NHID
    return (r == c).astype(F32)


def _hdot(a, b):
    return jnp.dot(a, b, preferred_element_type=F32,
                   precision=lax.Precision.HIGHEST)


def _normcaps(x, g, e):
    n2 = _hdot(x * x, g)
    inv = 1.0 / jnp.maximum(jnp.sqrt(n2), 1e-12)
    return x * _hdot(inv, e)


def _softmax8(p):
    m = jnp.max(p, axis=1, keepdims=True)
    ex = jnp.exp(p - m)
    return ex / jnp.sum(ex, axis=1, keepdims=True)


def _routing_patch(v0, v1, v2, v3, a0, a1, b0, b1, g, e):
    v = [v0, v1, v2, v3]
    beq = b0 == b1
    for _ in range(NLAYER):
        w = [_normcaps(vk, g, e) for vk in v]
        z0, z1 = w[0], w[1]
        ub0, ub1 = w[2], w[3]
        for _ in range(ROUTIT):
            p0 = _softmax8(_hdot(z0 * ub0, g))
            s0 = z0 * _hdot(p0, e)
            p1 = _softmax8(_hdot(z1 * ub1, g))
            s1 = z1 * _hdot(p1, e)
            n_same = _normcaps(w[2] + s0 + s1, g, e)
            n_b0 = _normcaps(w[2] + s0, g, e)
            n_b1 = _normcaps(w[3] + s1, g, e)
            ub0 = jnp.where(beq, n_same, n_b0)
            ub1 = jnp.where(beq, n_same, n_b1)
        v = [
            jnp.where(a0 == b0, ub0, jnp.where(a0 == b1, ub1, z0)),
            jnp.where(a1 == b0, ub0, jnp.where(a1 == b1, ub1, z1)),
            ub0,
            ub1,
        ]
    return v[2], v[3]


def _prep_body(s_ref, obj_ref, w1_ref, b1_ref, pred4_ref,
               a_ref, b_ref, patch_ref):
    obj = obj_ref[...]
    a_ref[...] = jnp.dot(obj, w1_ref[0:DREP, :], preferred_element_type=F32)
    b_ref[...] = jnp.dot(obj, w1_ref[DREP + NHID:, :],
                         preferred_element_type=F32)
    g, e = _gmat(), _emat()
    wp = w1_ref[DREP:DREP + NHID, :]
    vs = []
    for k in range(4):
        sr = s_ref[4 + 2 * k]
        orr = s_ref[5 + 2 * k]
        hrow = (a_ref[pl.ds(sr, 1), :] + b_ref[pl.ds(orr, 1), :]
                + jnp.dot(pred4_ref[k:k + 1, :], wp,
                          preferred_element_type=F32)
                + b1_ref[...])
        vs.append(_leaky(hrow))
    p0, p1 = _routing_patch(vs[0], vs[1], vs[2], vs[3],
                            s_ref[0], s_ref[1], s_ref[2], s_ref[3], g, e)
    patch_ref[...] = jnp.concatenate([p0, p1, jnp.zeros((6, DREP), F32)], 0)


def _prep_call(ints, obj_vecs, w1, b1r, pred4):
    grid_spec = pltpu.PrefetchScalarGridSpec(
        num_scalar_prefetch=1,
        grid=(1,),
        in_specs=[
            pl.BlockSpec((O, DREP), lambda i, s: (0, 0)),
            pl.BlockSpec((2 * DREP + NHID, DREP), lambda i, s: (0, 0)),
            pl.BlockSpec((1, DREP), lambda i, s: (0, 0)),
            pl.BlockSpec((8, NHID), lambda i, s: (0, 0)),
        ],
        out_specs=[
            pl.BlockSpec((O, DREP), lambda i, s: (0, 0)),
            pl.BlockSpec((O, DREP), lambda i, s: (0, 0)),
            pl.BlockSpec((8, DREP), lambda i, s: (0, 0)),
        ],
    )
    return pl.pallas_call(
        _prep_body,
        grid_spec=grid_spec,
        out_shape=[
            jax.ShapeDtypeStruct((O, DREP), F32),
            jax.ShapeDtypeStruct((O, DREP), F32),
            jax.ShapeDtypeStruct((8, DREP), F32),
        ],
    )(ints, obj_vecs, w1, b1r, pred4)


def _sc_gather(a, b, s_idx, o_idx):
    mesh = plsc.VectorSubcoreMesh(core_axis_name="c", subcore_axis_name="s")

    @functools.partial(
        pl.kernel,
        mesh=mesh,
        out_type=[
            jax.ShapeDtypeStruct((T, DREP), F32),
            jax.ShapeDtypeStruct((T, DREP), F32),
        ],
        scratch_types=[
            pltpu.VMEM((CH,), jnp.int32),
            pltpu.VMEM((CH,), jnp.int32),
            pltpu.VMEM((CH, DREP), F32),
            pltpu.VMEM((CH, DREP), F32),
            pltpu.SemaphoreType.DMA,
            pltpu.SemaphoreType.DMA,
        ],
    )
    def k(a_hbm, b_hbm, si_hbm, oi_hbm, ga_hbm, gb_hbm,
          si_v, oi_v, buf_a, buf_b, sem_a, sem_b):
        wid = lax.axis_index("s") * NC + lax.axis_index("c")
        nloop = (NCHUNK + NW - 1) // NW

        @pl.loop(0, nloop)
        def _(j):
            cid = j * NW + wid

            @pl.when(cid < NCHUNK)
            def _():
                base = cid * CH
                pltpu.sync_copy(si_hbm.at[pl.ds(base, CH)], si_v)
                pltpu.sync_copy(oi_hbm.at[pl.ds(base, CH)], oi_v)
                ca = pltpu.async_copy(a_hbm.at[si_v], buf_a, sem_a)
                cb = pltpu.async_copy(b_hbm.at[oi_v], buf_b, sem_b)
                ca.wait()
                cb.wait()
                pltpu.sync_copy(buf_a, ga_hbm.at[pl.ds(base, CH)])
                pltpu.sync_copy(buf_b, gb_hbm.at[pl.ds(base, CH)])

    return k(a, b, s_idx, o_idx)


def _pass1_body(s_ref, ga_ref, gb_ref, pred_ref, w1p_ref, b1_ref,
                patch_ref, x2_ref, acc_ref):
    i = pl.program_id(0)
    h = (ga_ref[...] + gb_ref[...]
         + jnp.dot(pred_ref[...], w1p_ref[...], preferred_element_type=F32)
         + b1_ref[...])
    h = _leaky(h)
    g, e = _gmat(), _emat()
    x2 = _normcaps(h, g, e)
    rows = i * TILE + lax.broadcasted_iota(jnp.int32, (TILE, 1), 0)
    x2 = jnp.where(rows == s_ref[2], patch_ref[0:1, :], x2)
    x2 = jnp.where(rows == s_ref[3], patch_ref[1:2, :], x2)
    x2_ref[...] = x2
    gram = lax.dot_general(x2, x2, (((0,), (0,)), ((), ())),
                           preferred_element_type=F32,
                           precision=lax.Precision.HIGHEST)

    @pl.when(i == 0)
    def _():
        acc_ref[...] = jnp.zeros((DREP + 8, DREP), F32)

    acc_ref[0:DREP, :] += gram
    acc_ref[DREP:DREP + 1, :] += jnp.sum(x2, axis=0, keepdims=True)


def _pass1_call(ints, ga, gb, pred_vecs, w1, b1r, patch):
    grid_spec = pltpu.PrefetchScalarGridSpec(
        num_scalar_prefetch=1,
        grid=(NT,),
        in_specs=[
            pl.BlockSpec((TILE, DREP), lambda i, s: (i, 0)),
            pl.BlockSpec((TILE, DREP), lambda i, s: (i, 0)),
            pl.BlockSpec((TILE, NHID), lambda i, s: (i, 0)),
            pl.BlockSpec((NHID, DREP), lambda i, s: (8, 0)),
            pl.BlockSpec((1, DREP), lambda i, s: (0, 0)),
            pl.BlockSpec((8, DREP), lambda i, s: (0, 0)),
        ],
        out_specs=[
            pl.BlockSpec((TILE, DREP), lambda i, s: (i, 0)),
            pl.BlockSpec((DREP + 8, DREP), lambda i, s: (0, 0)),
        ],
    )
    return pl.pallas_call(
        _pass1_body,
        grid_spec=grid_spec,
        out_shape=[
            jax.ShapeDtypeStruct((T, DREP), F32),
            jax.ShapeDtypeStruct((DREP + 8, DREP), F32),
        ],
    )(ints, ga, gb, pred_vecs, w1, b1r, patch)


def _pass2_body(x2_ref, acc_ref, cw_ref, cb_ref, g_ref, b_ref,
                outs_ref, outp_ref, outo_ref):
    cw = cw_ref[...]
    cb = cb_ref[...]
    y = jnp.dot(x2_ref[...], cw, preferred_element_type=F32) + cb
    csum = acc_ref[DREP:DREP + 1, :]
    m1 = jnp.dot(csum, cw, preferred_element_type=F32)
    mm = jnp.dot(acc_ref[0:DREP, :], cw, preferred_element_type=F32)
    diag = jnp.sum(cw * mm, axis=0, keepdims=True)
    mu = (m1 + T * cb) * (1.0 / T)
    ey2 = (diag + 2.0 * cb * m1 + T * cb * cb) * (1.0 / T)
    var = ey2 - mu * mu
    inv = 1.0 / jnp.sqrt(var + 1e-5)
    ob = _leaky((y - mu) * inv * g_ref[...] + b_ref[...])
    outs_ref[...] = ob[:, 0:H]
    outp_ref[...] = ob[:, H:H + DOUT]
    outo_ref[...] = ob[:, H + DOUT:]


def _pass2_call(x2, acc, cw1, cb1r, g1r, b1r):
    return pl.pallas_call(
        _pass2_body,
        grid=(NT,),
        in_specs=[
            pl.BlockSpec((TILE, DREP), lambda i: (i, 0)),
            pl.BlockSpec((DREP + 8, DREP), lambda i: (0, 0)),
            pl.BlockSpec((DREP, D1_OUT), lambda i: (0, 0)),
            pl.BlockSpec((1, D1_OUT), lambda i: (0, 0)),
            pl.BlockSpec((1, D1_OUT), lambda i: (0, 0)),
            pl.BlockSpec((1, D1_OUT), lambda i: (0, 0)),
        ],
        out_specs=[
            pl.BlockSpec((TILE, H), lambda i: (i, 0)),
            pl.BlockSpec((TILE, DOUT), lambda i: (i, 0)),
            pl.BlockSpec((TILE, H), lambda i: (i, 0)),
        ],
        out_shape=[
            jax.ShapeDtypeStruct((T, H), F32),
            jax.ShapeDtypeStruct((T, DOUT), F32),
            jax.ShapeDtypeStruct((T, H), F32),
        ],
    )(x2, acc, cw1, cb1r, g1r, b1r)


STRIPE = 624
TAIL = O - NS * STRIPE


def _sc_scatter(outs, outo, s_idx, o_idx, zrow):
    mesh = plsc.VectorSubcoreMesh(core_axis_name="c", subcore_axis_name="s")

    @functools.partial(
        pl.kernel,
        mesh=mesh,
        out_type=jax.ShapeDtypeStruct((NC * O, H), F32),
        scratch_types=[
            pltpu.VMEM((CHS,), jnp.int32),
            pltpu.VMEM((CHS, H), F32),
            pltpu.VMEM_SHARED((O, H), F32),
        ],
    )
    def k(outs_hbm, outo_hbm, si_hbm, oi_hbm, zrow_hbm,
          pp_hbm, idx_v, val_v, pool_sh):
        cid_core = lax.axis_index("c")
        sid = lax.axis_index("s")
        wid = sid * NC + cid_core

        pltpu.sync_copy(zrow_hbm, val_v)
        for t in range(SCHN + 1):
            sz = CHS if t < SCHN else SREM
            off = sid * STRIPE + t * CHS
            pltpu.sync_copy(val_v.at[pl.ds(0, sz)],
                            pool_sh.at[pl.ds(off, sz)])

        @pl.when(sid == 0)
        def _():
            pltpu.sync_copy(val_v.at[pl.ds(0, TAIL)],
                            pool_sh.at[pl.ds(NS * STRIPE, TAIL)])

        plsc.subcore_barrier()
        nloop = (NCHUNKS + NW - 1) // NW

        @pl.loop(0, nloop)
        def _(j):
            cid = j * NW + wid

            @pl.when(cid < NCHUNKS)
            def _():
                base = cid * CHS
                pltpu.sync_copy(si_hbm.at[pl.ds(base, CHS)], idx_v)
                pltpu.sync_copy(outs_hbm.at[pl.ds(base, CHS)], val_v)
                pltpu.sync_copy(val_v, pool_sh.at[idx_v], add=True)
                pltpu.sync_copy(oi_hbm.at[pl.ds(base, CHS)], idx_v)
                pltpu.sync_copy(outo_hbm.at[pl.ds(base, CHS)], val_v)
                pltpu.sync_copy(val_v, pool_sh.at[idx_v], add=True)

        plsc.subcore_barrier()

        for t in range(SCHN + 1):
            sz = CHS if t < SCHN else SREM
            soff = sid * STRIPE + t * CHS
            doff = cid_core * O + soff
            pltpu.sync_copy(pool_sh.at[pl.ds(soff, sz)],
                            val_v.at[pl.ds(0, sz)])
            pltpu.sync_copy(val_v.at[pl.ds(0, sz)],
                            pp_hbm.at[pl.ds(doff, sz)])

        @pl.when(sid == 0)
        def _():
            tbase = cid_core * O + NS * STRIPE
            pltpu.sync_copy(pool_sh.at[pl.ds(NS * STRIPE, TAIL)],
                            val_v.at[pl.ds(0, TAIL)])
            pltpu.sync_copy(val_v.at[pl.ds(0, TAIL)],
                            pp_hbm.at[pl.ds(tbase, TAIL)])

    return k(outs, outo, s_idx, o_idx, zrow)


def _sc_counts(s_idx, o_idx, zrow, ones):
    mesh = plsc.VectorSubcoreMesh(core_axis_name="c", subcore_axis_name="s")

    @functools.partial(
        pl.kernel,
        mesh=mesh,
        out_type=jax.ShapeDtypeStruct((NC * O, H), F32),
        scratch_types=[
            pltpu.VMEM((CHS,), jnp.int32),
            pltpu.VMEM((CHS, H), F32),
            pltpu.VMEM((CHS, H), F32),
            pltpu.VMEM_SHARED((O, H), F32),
        ],
    )
    def k(si_hbm, oi_hbm, zrow_hbm, ones_hbm, cc_hbm,
          idx_v, val_v, ones_v, cnt_sh):
        cid_core = lax.axis_index("c")
        sid = lax.axis_index("s")
        wid = sid * NC + cid_core

        pltpu.sync_copy(zrow_hbm, val_v)
        pltpu.sync_copy(ones_hbm, ones_v)
        for t in range(SCHN + 1):
            sz = CHS if t < SCHN else SREM
            off = sid * STRIPE + t * CHS
            pltpu.sync_copy(val_v.at[pl.ds(0, sz)],
                            cnt_sh.at[pl.ds(off, sz)])

        @pl.when(sid == 0)
        def _():
            pltpu.sync_copy(val_v.at[pl.ds(0, TAIL)],
                            cnt_sh.at[pl.ds(NS * STRIPE, TAIL)])

        plsc.subcore_barrier()
        nloop = (NCHUNKS + NW - 1) // NW

        @pl.loop(0, nloop)
        def _(j):
            cid = j * NW + wid

            @pl.when(cid < NCHUNKS)
            def _():
                base = cid * CHS
                pltpu.sync_copy(si_hbm.at[pl.ds(base, CHS)], idx_v)
                pltpu.sync_copy(ones_v, cnt_sh.at[idx_v], add=True)
                pltpu.sync_copy(oi_hbm.at[pl.ds(base, CHS)], idx_v)
                pltpu.sync_copy(ones_v, cnt_sh.at[idx_v], add=True)

        plsc.subcore_barrier()

        for t in range(SCHN + 1):
            sz = CHS if t < SCHN else SREM
            soff = sid * STRIPE + t * CHS
            doff = cid_core * O + soff
            pltpu.sync_copy(cnt_sh.at[pl.ds(soff, sz)],
                            val_v.at[pl.ds(0, sz)])
            pltpu.sync_copy(val_v.at[pl.ds(0, sz)],
                            cc_hbm.at[pl.ds(doff, sz)])

        @pl.when(sid == 0)
        def _():
            tbase = cid_core * O + NS * STRIPE
            pltpu.sync_copy(cnt_sh.at[pl.ds(NS * STRIPE, TAIL)],
                            val_v.at[pl.ds(0, TAIL)])
            pltpu.sync_copy(val_v.at[pl.ds(0, TAIL)],
                            cc_hbm.at[pl.ds(tbase, TAIL)])

    return k(s_idx, o_idx, zrow, ones)


def _final_body(s_ref, pp_ref, cc_ref, pw_ref, pb_ref, cw_ref, cb_ref,
                g2_ref, b2_ref, out_ref, scr_ref):
    pooled = pp_ref[0:O, :] + pp_ref[O:2 * O, :]
    cnt = cc_ref[0:O, 0:1] + cc_ref[O:2 * O, 0:1]
    c0 = jnp.maximum(cnt, 1.0)
    pavg = pooled / c0
    h = _leaky(jnp.dot(pavg, pw_ref[...], preferred_element_type=F32)
               + pb_ref[...])
    scr_ref[...] = h
    g, e = _gmat(), _emat()
    vs = [scr_ref[pl.ds(s_ref[k], 1), :] for k in range(4)]
    p0, p1 = _routing_patch(vs[0], vs[1], vs[2], vs[3],
                            s_ref[0], s_ref[1], s_ref[2], s_ref[3], g, e)
    x2 = _normcaps(h, g, e)
    scr_ref[...] = x2
    scr_ref[pl.ds(s_ref[2], 1), :] = p0
    scr_ref[pl.ds(s_ref[3], 1), :] = p1
    y = jnp.dot(scr_ref[...], cw_ref[...], preferred_element_type=F32) \
        + cb_ref[...]
    mu = jnp.mean(y, axis=0, keepdims=True)
    xc = y - mu
    var = jnp.mean(xc * xc, axis=0, keepdims=True)
    out_ref[...] = _leaky(xc * (1.0 / jnp.sqrt(var + 1e-5)) * g2_ref[...]
                          + b2_ref[...])


def _final_call(ints, pp, cc, pw2, pb2r, cw2, cb2r, g2r, b2r):
    grid_spec = pltpu.PrefetchScalarGridSpec(
        num_scalar_prefetch=1,
        grid=(1,),
        in_specs=[
            pl.BlockSpec((NC * O, H), lambda i, s: (0, 0)),
            pl.BlockSpec((NC * O, H), lambda i, s: (0, 0)),
            pl.BlockSpec((DREP, DREP), lambda i, s: (0, 0)),
            pl.BlockSpec((1, DREP), lambda i, s: (0, 0)),
            pl.BlockSpec((DREP, DOUT), lambda i, s: (0, 0)),
            pl.BlockSpec((1, DOUT), lambda i, s: (0, 0)),
            pl.BlockSpec((1, DOUT), lambda i, s: (0, 0)),
            pl.BlockSpec((1, DOUT), lambda i, s: (0, 0)),
        ],
        out_specs=pl.BlockSpec((O, DOUT), lambda i, s: (0, 0)),
        scratch_shapes=[pltpu.VMEM((O, DREP), F32)],
    )
    return pl.pallas_call(
        _final_body,
        grid_spec=grid_spec,
        out_shape=jax.ShapeDtypeStruct((O, DOUT), F32),
    )(ints, pp, cc, pw2, pb2r, cw2, cb2r, g2r, b2r)


def kernel(obj_vecs, pred_vecs, edges, pca_W1, pca_b1, clf_W1, clf_b1,
           bn1_gamma, bn1_beta, pca_W2, pca_b2, clf_W2, clf_b2,
           bn2_gamma, bn2_beta):
    s_idx = edges[:, 0]
    o_idx = edges[:, 1]
    idx4 = jnp.stack([edges[0, 0], edges[0, 1], edges[1, 0], edges[1, 1]])
    rows4 = edges[idx4]
    ints = jnp.concatenate([idx4, rows4.reshape(-1),
                            jnp.zeros((4,), jnp.int32)])
    pred4 = jnp.concatenate([pred_vecs[idx4], jnp.zeros((4, NHID), F32)], 0)

    zrow = jnp.zeros((CHS, H), F32)
    onesr = jnp.ones((CHS, H), F32)
    cc = _sc_counts(s_idx, o_idx, zrow, onesr)
    a, b, patch = _prep_call(ints, obj_vecs, pca_W1,
                             pca_b1.reshape(1, -1), pred4)
    ga, gb = _sc_gather(a, b, s_idx, o_idx)
    x2, acc = _pass1_call(ints, ga, gb, pred_vecs, pca_W1,
                          pca_b1.reshape(1, -1), patch)
    outs, outp, outo = _pass2_call(x2, acc, clf_W1, clf_b1.reshape(1, -1),
                                   bn1_gamma.reshape(1, -1),
                                   bn1_beta.reshape(1, -1))
    pp = _sc_scatter(outs, outo, s_idx, o_idx, zrow)
    obj_out = _final_call(ints, pp, cc, pca_W2, pca_b2.reshape(1, -1),
                          clf_W2, clf_b2.reshape(1, -1),
                          bn2_gamma.reshape(1, -1), bn2_beta.reshape(1, -1))
    return (obj_out, outp)


# speedup vs baseline: 1.2060x; 1.2060x over previous
import functools

import jax
import jax.numpy as jnp
from jax import lax
from jax.experimental import pallas as pl
from jax.experimental.pallas import tpu as pltpu
from jax.experimental.pallas import tpu_sc as plsc

O = 10000
T = 160000
H = 128
DOUT = 32
NCAPS = 8
NHID = 16
DREP = NCAPS * NHID
D1_OUT = 2 * H + DOUT
NLAYER = 2
ROUTIT = 3

TILE = 2000
NT = T // TILE

NC, NS = 2, 16
NW = NC * NS
CH = 128
NCHUNK = T // CH
HEXT = H + NHID
CHS = 64
NCHUNKS = T // CHS
SCHN = 624 // CHS
SREM = 624 - SCHN * CHS
F32 = jnp.float32


def _leaky(x):
    return jnp.where(x >= 0, x, 0.01 * x)


def _gmat():
    r = lax.broadcasted_iota(jnp.int32, (DREP, NCAPS), 0) // NHID
    c = lax.broadcasted_iota(jnp.int32, (DREP, NCAPS), 1)
    return (r == c).astype(F32)


def _emat():
    r = lax.broadcasted_iota(jnp.int32, (NCAPS, DREP), 0)
    c = lax.broadcasted_iota(jnp.int32, (NCAPS, DREP), 1) // NHID
    return (r == c).astype(F32)


def _hdot(a, b):
    return jnp.dot(a, b, preferred_element_type=F32,
                   precision=lax.Precision.HIGHEST)


def _normcaps(x, g, e):
    n2 = _hdot(x * x, g)
    inv = 1.0 / jnp.maximum(jnp.sqrt(n2), 1e-12)
    return x * _hdot(inv, e)


def _softmax8(p):
    m = jnp.max(p, axis=1, keepdims=True)
    ex = jnp.exp(p - m)
    return ex / jnp.sum(ex, axis=1, keepdims=True)


def _routing_patch(v0, v1, v2, v3, a0, a1, b0, b1, g, e):
    v = [v0, v1, v2, v3]
    beq = b0 == b1
    for _ in range(NLAYER):
        w = [_normcaps(vk, g, e) for vk in v]
        z0, z1 = w[0], w[1]
        ub0, ub1 = w[2], w[3]
        for _ in range(ROUTIT):
            p0 = _softmax8(_hdot(z0 * ub0, g))
            s0 = z0 * _hdot(p0, e)
            p1 = _softmax8(_hdot(z1 * ub1, g))
            s1 = z1 * _hdot(p1, e)
            n_same = _normcaps(w[2] + s0 + s1, g, e)
            n_b0 = _normcaps(w[2] + s0, g, e)
            n_b1 = _normcaps(w[3] + s1, g, e)
            ub0 = jnp.where(beq, n_same, n_b0)
            ub1 = jnp.where(beq, n_same, n_b1)
        v = [
            jnp.where(a0 == b0, ub0, jnp.where(a0 == b1, ub1, z0)),
            jnp.where(a1 == b0, ub0, jnp.where(a1 == b1, ub1, z1)),
            ub0,
            ub1,
        ]
    return v[2], v[3]


def _prep_body(s_ref, obj_ref, w1_ref, b1_ref, pred4_ref,
               a_ref, b_ref, patch_ref):
    obj = obj_ref[...]
    a_ref[...] = jnp.dot(obj, w1_ref[0:DREP, :], preferred_element_type=F32)
    b_ref[...] = jnp.dot(obj, w1_ref[DREP + NHID:, :],
                         preferred_element_type=F32)
    g, e = _gmat(), _emat()
    wp = w1_ref[DREP:DREP + NHID, :]
    vs = []
    for k in range(4):
        sr = s_ref[4 + 2 * k]
        orr = s_ref[5 + 2 * k]
        hrow = (a_ref[pl.ds(sr, 1), :] + b_ref[pl.ds(orr, 1), :]
                + jnp.dot(pred4_ref[k:k + 1, :], wp,
                          preferred_element_type=F32)
                + b1_ref[...])
        vs.append(_leaky(hrow))
    p0, p1 = _routing_patch(vs[0], vs[1], vs[2], vs[3],
                            s_ref[0], s_ref[1], s_ref[2], s_ref[3], g, e)
    patch_ref[...] = jnp.concatenate([p0, p1, jnp.zeros((6, DREP), F32)], 0)


def _prep_call(ints, obj_vecs, w1, b1r, pred4):
    grid_spec = pltpu.PrefetchScalarGridSpec(
        num_scalar_prefetch=1,
        grid=(1,),
        in_specs=[
            pl.BlockSpec((O, DREP), lambda i, s: (0, 0)),
            pl.BlockSpec((2 * DREP + NHID, DREP), lambda i, s: (0, 0)),
            pl.BlockSpec((1, DREP), lambda i, s: (0, 0)),
            pl.BlockSpec((8, NHID), lambda i, s: (0, 0)),
        ],
        out_specs=[
            pl.BlockSpec((O, DREP), lambda i, s: (0, 0)),
            pl.BlockSpec((O, DREP), lambda i, s: (0, 0)),
            pl.BlockSpec((8, DREP), lambda i, s: (0, 0)),
        ],
    )
    return pl.pallas_call(
        _prep_body,
        grid_spec=grid_spec,
        out_shape=[
            jax.ShapeDtypeStruct((O, DREP), F32),
            jax.ShapeDtypeStruct((O, DREP), F32),
            jax.ShapeDtypeStruct((8, DREP), F32),
        ],
    )(ints, obj_vecs, w1, b1r, pred4)


def _sc_gather(a, b, s_idx, o_idx):
    mesh = plsc.VectorSubcoreMesh(core_axis_name="c", subcore_axis_name="s")

    @functools.partial(
        pl.kernel,
        mesh=mesh,
        out_type=[
            jax.ShapeDtypeStruct((T, DREP), F32),
            jax.ShapeDtypeStruct((T, DREP), F32),
        ],
        scratch_types=[
            pltpu.VMEM((2, CH), jnp.int32),
            pltpu.VMEM((2, CH), jnp.int32),
            pltpu.VMEM((CH, DREP), F32),
            pltpu.VMEM((CH, DREP), F32),
            pltpu.VMEM((CH, DREP), F32),
            pltpu.VMEM((CH, DREP), F32),
            pltpu.SemaphoreType.DMA,
            pltpu.SemaphoreType.DMA,
            pltpu.SemaphoreType.DMA,
            pltpu.SemaphoreType.DMA,
            pltpu.SemaphoreType.DMA,
            pltpu.SemaphoreType.DMA,
        ],
    )
    def k(a_hbm, b_hbm, si_hbm, oi_hbm, ga_hbm, gb_hbm,
          si_v, oi_v, buf_a0, buf_a1, buf_b0, buf_b1,
          sem_i0, sem_i1, sem_g0, sem_g1, sem_w0, sem_w1):
        wid = lax.axis_index("s") * NC + lax.axis_index("c")
        nloop = (NCHUNK + NW - 1) // NW
        bufs = ((buf_a0, buf_b0, sem_i0, sem_g0, sem_w0),
                (buf_a1, buf_b1, sem_i1, sem_g1, sem_w1))

        def cid(j):
            return j * NW + wid

        def issue_idx(j, p):
            base = cid(j) * CH
            sem_i = bufs[p][2]
            pltpu.async_copy(si_hbm.at[pl.ds(base, CH)], si_v.at[p], sem_i)
            pltpu.async_copy(oi_hbm.at[pl.ds(base, CH)], oi_v.at[p], sem_i)

        def wait_idx(j, p):
            base = cid(j) * CH
            sem_i = bufs[p][2]
            pltpu.make_async_copy(si_hbm.at[pl.ds(base, CH)], si_v.at[p],
                                  sem_i).wait()
            pltpu.make_async_copy(oi_hbm.at[pl.ds(base, CH)], oi_v.at[p],
                                  sem_i).wait()

        @pl.when(cid(0) < NCHUNK)
        def _():
            issue_idx(0, 0)

        @pl.loop(0, nloop)
        def _(j):
            p = (j % 2).astype(jnp.int32) if hasattr(j % 2, "astype") else j % 2
            for par in range(2):
                @pl.when(((j % 2) == par) & (cid(j) < NCHUNK))
                def _(par=par):
                    buf_a, buf_b, sem_i, sem_g, sem_w = bufs[par]
                    wait_idx(j, par)

                    @pl.when(j >= 2)
                    def _():
                        pltpu.make_async_copy(
                            buf_a, ga_hbm.at[pl.ds(cid(j - 2) * CH, CH)],
                            sem_w).wait()
                        pltpu.make_async_copy(
                            buf_b, gb_hbm.at[pl.ds(cid(j - 2) * CH, CH)],
                            sem_w).wait()

                    pltpu.async_copy(a_hbm.at[si_v.at[par]], buf_a, sem_g)
                    pltpu.async_copy(b_hbm.at[oi_v.at[par]], buf_b, sem_g)

                    @pl.when(cid(j + 1) < NCHUNK)
                    def _():
                        issue_idx(j + 1, 1 - par)

                    pltpu.make_async_copy(a_hbm.at[si_v.at[par]], buf_a,
                                          sem_g).wait()
                    pltpu.make_async_copy(b_hbm.at[oi_v.at[par]], buf_b,
                                          sem_g).wait()
                    base = cid(j) * CH
                    pltpu.async_copy(buf_a, ga_hbm.at[pl.ds(base, CH)],
                                     sem_w)
                    pltpu.async_copy(buf_b, gb_hbm.at[pl.ds(base, CH)],
                                     sem_w)

        for par in range(2):
            jlast = nloop - 1 - ((nloop - 1 + par) % 2)

            @pl.when(cid(jlast) < NCHUNK)
            def _(par=par, jlast=jlast):
                buf_a, buf_b, _, _, sem_w = bufs[par]
                pltpu.make_async_copy(
                    buf_a, ga_hbm.at[pl.ds(cid(jlast) * CH, CH)],
                    sem_w).wait()
                pltpu.make_async_copy(
                    buf_b, gb_hbm.at[pl.ds(cid(jlast) * CH, CH)],
                    sem_w).wait()

    return k(a, b, s_idx, o_idx)


def _pass1_body(s_ref, ga_ref, gb_ref, pred_ref, w1p_ref, b1_ref,
                cw_ref, cb_ref, patch_ref, x2_ref, acc_ref):
    i = pl.program_id(0)
    h = (ga_ref[...] + gb_ref[...]
         + jnp.dot(pred_ref[...], w1p_ref[...], preferred_element_type=F32)
         + b1_ref[...])
    h = _leaky(h)
    g, e = _gmat(), _emat()
    x2 = _normcaps(h, g, e)
    rows = i * TILE + lax.broadcasted_iota(jnp.int32, (TILE, 1), 0)
    x2 = jnp.where(rows == s_ref[2], patch_ref[0:1, :], x2)
    x2 = jnp.where(rows == s_ref[3], patch_ref[1:2, :], x2)
    x2_ref[...] = x2
    y = jnp.dot(x2, cw_ref[...], preferred_element_type=F32) + cb_ref[...]

    @pl.when(i == 0)
    def _():
        acc_ref[...] = jnp.zeros((8, D1_OUT), F32)

    acc_ref[0:1, :] += jnp.sum(y, axis=0, keepdims=True)
    acc_ref[1:2, :] += jnp.sum(y * y, axis=0, keepdims=True)


def _pass1_call(ints, ga, gb, pred_vecs, w1, b1r, cw1, cb1r, patch):
    grid_spec = pltpu.PrefetchScalarGridSpec(
        num_scalar_prefetch=1,
        grid=(NT,),
        in_specs=[
            pl.BlockSpec((TILE, DREP), lambda i, s: (i, 0)),
            pl.BlockSpec((TILE, DREP), lambda i, s: (i, 0)),
            pl.BlockSpec((TILE, NHID), lambda i, s: (i, 0)),
            pl.BlockSpec((NHID, DREP), lambda i, s: (8, 0)),
            pl.BlockSpec((1, DREP), lambda i, s: (0, 0)),
            pl.BlockSpec((DREP, D1_OUT), lambda i, s: (0, 0)),
            pl.BlockSpec((1, D1_OUT), lambda i, s: (0, 0)),
            pl.BlockSpec((8, DREP), lambda i, s: (0, 0)),
        ],
        out_specs=[
            pl.BlockSpec((TILE, DREP), lambda i, s: (i, 0)),
            pl.BlockSpec((8, D1_OUT), lambda i, s: (0, 0)),
        ],
    )
    return pl.pallas_call(
        _pass1_body,
        grid_spec=grid_spec,
        out_shape=[
            jax.ShapeDtypeStruct((T, DREP), F32),
            jax.ShapeDtypeStruct((8, D1_OUT), F32),
        ],
    )(ints, ga, gb, pred_vecs, w1, b1r, cw1, cb1r, patch)


def _pass2_body(x2_ref, acc_ref, cw_ref, cb_ref, g_ref, b_ref,
                outs_ref, outp_ref, outo_ref):
    y = jnp.dot(x2_ref[...], cw_ref[...], preferred_element_type=F32) \
        + cb_ref[...]
    mu = acc_ref[0:1, :] * (1.0 / T)
    ey2 = acc_ref[1:2, :] * (1.0 / T)
    var = ey2 - mu * mu
    inv = 1.0 / jnp.sqrt(var + 1e-5)
    ob = _leaky((y - mu) * inv * g_ref[...] + b_ref[...])
    outs_ref[...] = ob[:, 0:H]
    outp_ref[...] = ob[:, H:H + DOUT]
    outo_ref[...] = ob[:, H + DOUT:]


def _pass2_call(x2, acc, cw1, cb1r, g1r, b1r):
    return pl.pallas_call(
        _pass2_body,
        grid=(NT,),
        in_specs=[
            pl.BlockSpec((TILE, DREP), lambda i: (i, 0)),
            pl.BlockSpec((8, D1_OUT), lambda i: (0, 0)),
            pl.BlockSpec((DREP, D1_OUT), lambda i: (0, 0)),
            pl.BlockSpec((1, D1_OUT), lambda i: (0, 0)),
            pl.BlockSpec((1, D1_OUT), lambda i: (0, 0)),
            pl.BlockSpec((1, D1_OUT), lambda i: (0, 0)),
        ],
        out_specs=[
            pl.BlockSpec((TILE, H), lambda i: (i, 0)),
            pl.BlockSpec((TILE, DOUT), lambda i: (i, 0)),
            pl.BlockSpec((TILE, H), lambda i: (i, 0)),
        ],
        out_shape=[
            jax.ShapeDtypeStruct((T, H), F32),
            jax.ShapeDtypeStruct((T, DOUT), F32),
            jax.ShapeDtypeStruct((T, H), F32),
        ],
    )(x2, acc, cw1, cb1r, g1r, b1r)


STRIPE = 624
TAIL = O - NS * STRIPE


def _sc_scatter(outs, outo, s_idx, o_idx, zrow):
    mesh = plsc.VectorSubcoreMesh(core_axis_name="c", subcore_axis_name="s")

    @functools.partial(
        pl.kernel,
        mesh=mesh,
        out_type=jax.ShapeDtypeStruct((NC * O, H), F32),
        scratch_types=[
            pltpu.VMEM((2, CHS), jnp.int32),
            pltpu.VMEM((2, CHS), jnp.int32),
            pltpu.VMEM((CHS, H), F32),
            pltpu.VMEM((CHS, H), F32),
            pltpu.VMEM((CHS, H), F32),
            pltpu.VMEM((CHS, H), F32),
            pltpu.VMEM_SHARED((O, H), F32),
            pltpu.SemaphoreType.DMA,
            pltpu.SemaphoreType.DMA,
        ],
    )
    def k(outs_hbm, outo_hbm, si_hbm, oi_hbm, zrow_hbm,
          pp_hbm, si_v, oi_v, vs0, vs1, vo0, vo1, pool_sh,
          sem_l0, sem_l1):
        cid_core = lax.axis_index("c")
        sid = lax.axis_index("s")
        wid = sid * NC + cid_core
        bufs = ((vs0, vo0, sem_l0), (vs1, vo1, sem_l1))

        pltpu.sync_copy(zrow_hbm, vs0)
        for t in range(SCHN + 1):
            sz = CHS if t < SCHN else SREM
            off = sid * STRIPE + t * CHS
            pltpu.sync_copy(vs0.at[pl.ds(0, sz)],
                            pool_sh.at[pl.ds(off, sz)])

        @pl.when(sid == 0)
        def _():
            pltpu.sync_copy(vs0.at[pl.ds(0, TAIL)],
                            pool_sh.at[pl.ds(NS * STRIPE, TAIL)])

        plsc.subcore_barrier()
        nloop = (NCHUNKS + NW - 1) // NW

        def cid(j):
            return j * NW + wid

        def issue_loads(j, p):
            base = cid(j) * CHS
            vs, vo, sem_l = bufs[p]
            pltpu.async_copy(si_hbm.at[pl.ds(base, CHS)], si_v.at[p], sem_l)
            pltpu.async_copy(outs_hbm.at[pl.ds(base, CHS)], vs, sem_l)
            pltpu.async_copy(oi_hbm.at[pl.ds(base, CHS)], oi_v.at[p], sem_l)
            pltpu.async_copy(outo_hbm.at[pl.ds(base, CHS)], vo, sem_l)

        def wait_loads(j, p):
            base = cid(j) * CHS
            vs, vo, sem_l = bufs[p]
            pltpu.make_async_copy(si_hbm.at[pl.ds(base, CHS)], si_v.at[p],
                                  sem_l).wait()
            pltpu.make_async_copy(outs_hbm.at[pl.ds(base, CHS)], vs,
                                  sem_l).wait()
            pltpu.make_async_copy(oi_hbm.at[pl.ds(base, CHS)], oi_v.at[p],
                                  sem_l).wait()
            pltpu.make_async_copy(outo_hbm.at[pl.ds(base, CHS)], vo,
                                  sem_l).wait()

        @pl.when(cid(0) < NCHUNKS)
        def _():
            issue_loads(0, 0)

        @pl.loop(0, nloop)
        def _(j):
            for par in range(2):
                @pl.when(((j % 2) == par) & (cid(j) < NCHUNKS))
                def _(par=par):
                    vs, vo, _ = bufs[par]
                    wait_loads(j, par)

                    @pl.when(cid(j + 1) < NCHUNKS)
                    def _():
                        issue_loads(j + 1, 1 - par)

                    pltpu.sync_copy(vs, pool_sh.at[si_v.at[par]], add=True)
                    pltpu.sync_copy(vo, pool_sh.at[oi_v.at[par]], add=True)

        plsc.subcore_barrier()

        for t in range(SCHN + 1):
            sz = CHS if t < SCHN else SREM
            soff = sid * STRIPE + t * CHS
            doff = cid_core * O + soff
            pltpu.sync_copy(pool_sh.at[pl.ds(soff, sz)],
                            vs0.at[pl.ds(0, sz)])
            pltpu.sync_copy(vs0.at[pl.ds(0, sz)],
                            pp_hbm.at[pl.ds(doff, sz)])

        @pl.when(sid == 0)
        def _():
            tbase = cid_core * O + NS * STRIPE
            pltpu.sync_copy(pool_sh.at[pl.ds(NS * STRIPE, TAIL)],
                            vo0.at[pl.ds(0, TAIL)])
            pltpu.sync_copy(vo0.at[pl.ds(0, TAIL)],
                            pp_hbm.at[pl.ds(tbase, TAIL)])

    return k(outs, outo, s_idx, o_idx, zrow)


def _sc_counts(s_idx, o_idx, zrow, ones):
    mesh = plsc.VectorSubcoreMesh(core_axis_name="c", subcore_axis_name="s")

    @functools.partial(
        pl.kernel,
        mesh=mesh,
        out_type=jax.ShapeDtypeStruct((NC * O, H), F32),
        scratch_types=[
            pltpu.VMEM((2, CHS), jnp.int32),
            pltpu.VMEM((2, CHS), jnp.int32),
            pltpu.VMEM((CHS, H), F32),
            pltpu.VMEM((CHS, H), F32),
            pltpu.VMEM_SHARED((O, H), F32),
            pltpu.SemaphoreType.DMA,
            pltpu.SemaphoreType.DMA,
        ],
    )
    def k(si_hbm, oi_hbm, zrow_hbm, ones_hbm, cc_hbm,
          si_v, oi_v, val_v, ones_v, cnt_sh, sem_l0, sem_l1):
        cid_core = lax.axis_index("c")
        sid = lax.axis_index("s")
        wid = sid * NC + cid_core
        sems = (sem_l0, sem_l1)

        pltpu.sync_copy(zrow_hbm, val_v)
        pltpu.sync_copy(ones_hbm, ones_v)
        for t in range(SCHN + 1):
            sz = CHS if t < SCHN else SREM
            off = sid * STRIPE + t * CHS
            pltpu.sync_copy(val_v.at[pl.ds(0, sz)],
                            cnt_sh.at[pl.ds(off, sz)])

        @pl.when(sid == 0)
        def _():
            pltpu.sync_copy(val_v.at[pl.ds(0, TAIL)],
                            cnt_sh.at[pl.ds(NS * STRIPE, TAIL)])

        plsc.subcore_barrier()
        nloop = (NCHUNKS + NW - 1) // NW

        def cid(j):
            return j * NW + wid

        def issue_idx(j, p):
            base = cid(j) * CHS
            pltpu.async_copy(si_hbm.at[pl.ds(base, CHS)], si_v.at[p],
                             sems[p])
            pltpu.async_copy(oi_hbm.at[pl.ds(base, CHS)], oi_v.at[p],
                             sems[p])

        def wait_idx(j, p):
            base = cid(j) * CHS
            pltpu.make_async_copy(si_hbm.at[pl.ds(base, CHS)], si_v.at[p],
                                  sems[p]).wait()
            pltpu.make_async_copy(oi_hbm.at[pl.ds(base, CHS)], oi_v.at[p],
                                  sems[p]).wait()

        @pl.when(cid(0) < NCHUNKS)
        def _():
            issue_idx(0, 0)

        @pl.loop(0, nloop)
        def _(j):
            for par in range(2):
                @pl.when(((j % 2) == par) & (cid(j) < NCHUNKS))
                def _(par=par):
                    wait_idx(j, par)

                    @pl.when(cid(j + 1) < NCHUNKS)
                    def _():
                        issue_idx(j + 1, 1 - par)

                    pltpu.sync_copy(ones_v, cnt_sh.at[si_v.at[par]],
                                    add=True)
                    pltpu.sync_copy(ones_v, cnt_sh.at[oi_v.at[par]],
                                    add=True)

        plsc.subcore_barrier()

        for t in range(SCHN + 1):
            sz = CHS if t < SCHN else SREM
            soff = sid * STRIPE + t * CHS
            doff = cid_core * O + soff
            pltpu.sync_copy(cnt_sh.at[pl.ds(soff, sz)],
                            val_v.at[pl.ds(0, sz)])
            pltpu.sync_copy(val_v.at[pl.ds(0, sz)],
                            cc_hbm.at[pl.ds(doff, sz)])

        @pl.when(sid == 0)
        def _():
            tbase = cid_core * O + NS * STRIPE
            pltpu.sync_copy(cnt_sh.at[pl.ds(NS * STRIPE, TAIL)],
                            val_v.at[pl.ds(0, TAIL)])
            pltpu.sync_copy(val_v.at[pl.ds(0, TAIL)],
                            cc_hbm.at[pl.ds(tbase, TAIL)])

    return k(s_idx, o_idx, zrow, ones)


def _final_body(s_ref, pp_ref, cc_ref, pw_ref, pb_ref, cw_ref, cb_ref,
                g2_ref, b2_ref, out_ref, scr_ref):
    pooled = pp_ref[0:O, :] + pp_ref[O:2 * O, :]
    cnt = cc_ref[0:O, 0:1] + cc_ref[O:2 * O, 0:1]
    c0 = jnp.maximum(cnt, 1.0)
    pavg = pooled / c0
    h = _leaky(jnp.dot(pavg, pw_ref[...], preferred_element_type=F32)
               + pb_ref[...])
    scr_ref[...] = h
    g, e = _gmat(), _emat()
    vs = [scr_ref[pl.ds(s_ref[k], 1), :] for k in range(4)]
    p0, p1 = _routing_patch(vs[0], vs[1], vs[2], vs[3],
                            s_ref[0], s_ref[1], s_ref[2], s_ref[3], g, e)
    x2 = _normcaps(h, g, e)
    scr_ref[...] = x2
    scr_ref[pl.ds(s_ref[2], 1), :] = p0
    scr_ref[pl.ds(s_ref[3], 1), :] = p1
    y = jnp.dot(scr_ref[...], cw_ref[...], preferred_element_type=F32) \
        + cb_ref[...]
    mu = jnp.mean(y, axis=0, keepdims=True)
    xc = y - mu
    var = jnp.mean(xc * xc, axis=0, keepdims=True)
    out_ref[...] = _leaky(xc * (1.0 / jnp.sqrt(var + 1e-5)) * g2_ref[...]
                          + b2_ref[...])


def _final_call(ints, pp, cc, pw2, pb2r, cw2, cb2r, g2r, b2r):
    grid_spec = pltpu.PrefetchScalarGridSpec(
        num_scalar_prefetch=1,
        grid=(1,),
        in_specs=[
            pl.BlockSpec((NC * O, H), lambda i, s: (0, 0)),
            pl.BlockSpec((NC * O, H), lambda i, s: (0, 0)),
            pl.BlockSpec((DREP, DREP), lambda i, s: (0, 0)),
            pl.BlockSpec((1, DREP), lambda i, s: (0, 0)),
            pl.BlockSpec((DREP, DOUT), lambda i, s: (0, 0)),
            pl.BlockSpec((1, DOUT), lambda i, s: (0, 0)),
            pl.BlockSpec((1, DOUT), lambda i, s: (0, 0)),
            pl.BlockSpec((1, DOUT), lambda i, s: (0, 0)),
        ],
        out_specs=pl.BlockSpec((O, DOUT), lambda i, s: (0, 0)),
        scratch_shapes=[pltpu.VMEM((O, DREP), F32)],
    )
    return pl.pallas_call(
        _final_body,
        grid_spec=grid_spec,
        out_shape=jax.ShapeDtypeStruct((O, DOUT), F32),
    )(ints, pp, cc, pw2, pb2r, cw2, cb2r, g2r, b2r)


def kernel(obj_vecs, pred_vecs, edges, pca_W1, pca_b1, clf_W1, clf_b1,
           bn1_gamma, bn1_beta, pca_W2, pca_b2, clf_W2, clf_b2,
           bn2_gamma, bn2_beta):
    s_idx = edges[:, 0]
    o_idx = edges[:, 1]
    idx4 = jnp.stack([edges[0, 0], edges[0, 1], edges[1, 0], edges[1, 1]])
    rows4 = edges[idx4]
    ints = jnp.concatenate([idx4, rows4.reshape(-1),
                            jnp.zeros((4,), jnp.int32)])
    pred4 = jnp.concatenate([pred_vecs[idx4], jnp.zeros((4, NHID), F32)], 0)

    zrow = jnp.zeros((CHS, H), F32)
    onesr = jnp.ones((CHS, H), F32)
    cc = _sc_counts(s_idx, o_idx, zrow, onesr)
    a, b, patch = _prep_call(ints, obj_vecs, pca_W1,
                             pca_b1.reshape(1, -1), pred4)
    ga, gb = _sc_gather(a, b, s_idx, o_idx)
    x2, acc = _pass1_call(ints, ga, gb, pred_vecs, pca_W1,
                          pca_b1.reshape(1, -1), clf_W1,
                          clf_b1.reshape(1, -1), patch)
    outs, outp, outo = _pass2_call(x2, acc, clf_W1, clf_b1.reshape(1, -1),
                                   bn1_gamma.reshape(1, -1),
                                   bn1_beta.reshape(1, -1))
    pp = _sc_scatter(outs, outo, s_idx, o_idx, zrow)
    obj_out = _final_call(ints, pp, cc, pca_W2, pca_b2.reshape(1, -1),
                          clf_W2, clf_b2.reshape(1, -1),
                          bn2_gamma.reshape(1, -1), bn2_beta.reshape(1, -1))
    return (obj_out, outp)


# speedup vs baseline: 1.2787x; 1.0603x over previous
import functools

import jax
import jax.numpy as jnp
from jax import lax
from jax.experimental import pallas as pl
from jax.experimental.pallas import tpu as pltpu
from jax.experimental.pallas import tpu_sc as plsc

O = 10000
T = 160000
H = 128
DOUT = 32
NCAPS = 8
NHID = 16
DREP = NCAPS * NHID
D1_OUT = 2 * H + DOUT
NLAYER = 2
ROUTIT = 3

TILE = 4000
NT = T // TILE

NC, NS = 2, 16
NW = NC * NS
CH = 128
NCHUNK = T // CH
HEXT = H + NHID
CHS = 64
NCHUNKS = T // CHS
SCHN = 624 // CHS
SREM = 624 - SCHN * CHS
F32 = jnp.float32


def _leaky(x):
    return jnp.where(x >= 0, x, 0.01 * x)


def _gmat():
    r = lax.broadcasted_iota(jnp.int32, (DREP, NCAPS), 0) // NHID
    c = lax.broadcasted_iota(jnp.int32, (DREP, NCAPS), 1)
    return (r == c).astype(F32)


def _emat():
    r = lax.broadcasted_iota(jnp.int32, (NCAPS, DREP), 0)
    c = lax.broadcasted_iota(jnp.int32, (NCAPS, DREP), 1) // NHID
    return (r == c).astype(F32)


def _hdot(a, b):
    return jnp.dot(a, b, preferred_element_type=F32,
                   precision=lax.Precision.HIGHEST)


def _normcaps(x, g, e):
    n2 = _hdot(x * x, g)
    inv = 1.0 / jnp.maximum(jnp.sqrt(n2), 1e-12)
    return x * _hdot(inv, e)


def _softmax8(p):
    m = jnp.max(p, axis=1, keepdims=True)
    ex = jnp.exp(p - m)
    return ex / jnp.sum(ex, axis=1, keepdims=True)


def _routing_patch(v0, v1, v2, v3, a0, a1, b0, b1, g, e):
    v = [v0, v1, v2, v3]
    beq = b0 == b1
    for _ in range(NLAYER):
        w = [_normcaps(vk, g, e) for vk in v]
        z0, z1 = w[0], w[1]
        ub0, ub1 = w[2], w[3]
        for _ in range(ROUTIT):
            p0 = _softmax8(_hdot(z0 * ub0, g))
            s0 = z0 * _hdot(p0, e)
            p1 = _softmax8(_hdot(z1 * ub1, g))
            s1 = z1 * _hdot(p1, e)
            n_same = _normcaps(w[2] + s0 + s1, g, e)
            n_b0 = _normcaps(w[2] + s0, g, e)
            n_b1 = _normcaps(w[3] + s1, g, e)
            ub0 = jnp.where(beq, n_same, n_b0)
            ub1 = jnp.where(beq, n_same, n_b1)
        v = [
            jnp.where(a0 == b0, ub0, jnp.where(a0 == b1, ub1, z0)),
            jnp.where(a1 == b0, ub0, jnp.where(a1 == b1, ub1, z1)),
            ub0,
            ub1,
        ]
    return v[2], v[3]


def _prep_body(s_ref, obj_ref, w1_ref, b1_ref, pred4_ref,
               a_ref, b_ref, patch_ref):
    obj = obj_ref[...]
    a_ref[...] = jnp.dot(obj, w1_ref[0:DREP, :], preferred_element_type=F32)
    b_ref[...] = jnp.dot(obj, w1_ref[DREP + NHID:, :],
                         preferred_element_type=F32)
    g, e = _gmat(), _emat()
    wp = w1_ref[DREP:DREP + NHID, :]
    vs = []
    for k in range(4):
        sr = s_ref[4 + 2 * k]
        orr = s_ref[5 + 2 * k]
        hrow = (a_ref[pl.ds(sr, 1), :] + b_ref[pl.ds(orr, 1), :]
                + jnp.dot(pred4_ref[k:k + 1, :], wp,
                          preferred_element_type=F32)
                + b1_ref[...])
        vs.append(_leaky(hrow))
    p0, p1 = _routing_patch(vs[0], vs[1], vs[2], vs[3],
                            s_ref[0], s_ref[1], s_ref[2], s_ref[3], g, e)
    patch_ref[...] = jnp.concatenate([p0, p1, jnp.zeros((6, DREP), F32)], 0)


def _prep_call(ints, obj_vecs, w1, b1r, pred4):
    grid_spec = pltpu.PrefetchScalarGridSpec(
        num_scalar_prefetch=1,
        grid=(1,),
        in_specs=[
            pl.BlockSpec((O, DREP), lambda i, s: (0, 0)),
            pl.BlockSpec((2 * DREP + NHID, DREP), lambda i, s: (0, 0)),
            pl.BlockSpec((1, DREP), lambda i, s: (0, 0)),
            pl.BlockSpec((8, NHID), lambda i, s: (0, 0)),
        ],
        out_specs=[
            pl.BlockSpec((O, DREP), lambda i, s: (0, 0)),
            pl.BlockSpec((O, DREP), lambda i, s: (0, 0)),
            pl.BlockSpec((8, DREP), lambda i, s: (0, 0)),
        ],
    )
    return pl.pallas_call(
        _prep_body,
        grid_spec=grid_spec,
        out_shape=[
            jax.ShapeDtypeStruct((O, DREP), F32),
            jax.ShapeDtypeStruct((O, DREP), F32),
            jax.ShapeDtypeStruct((8, DREP), F32),
        ],
    )(ints, obj_vecs, w1, b1r, pred4)


def _sc_gather(a, b, s_idx, o_idx):
    mesh = plsc.VectorSubcoreMesh(core_axis_name="c", subcore_axis_name="s")

    @functools.partial(
        pl.kernel,
        mesh=mesh,
        out_type=[
            jax.ShapeDtypeStruct((T, DREP), F32),
            jax.ShapeDtypeStruct((T, DREP), F32),
        ],
        scratch_types=[
            pltpu.VMEM((2, CH), jnp.int32),
            pltpu.VMEM((2, CH), jnp.int32),
            pltpu.VMEM((CH, DREP), F32),
            pltpu.VMEM((CH, DREP), F32),
            pltpu.VMEM((CH, DREP), F32),
            pltpu.VMEM((CH, DREP), F32),
            pltpu.SemaphoreType.DMA,
            pltpu.SemaphoreType.DMA,
            pltpu.SemaphoreType.DMA,
            pltpu.SemaphoreType.DMA,
            pltpu.SemaphoreType.DMA,
            pltpu.SemaphoreType.DMA,
        ],
    )
    def k(a_hbm, b_hbm, si_hbm, oi_hbm, ga_hbm, gb_hbm,
          si_v, oi_v, buf_a0, buf_a1, buf_b0, buf_b1,
          sem_i0, sem_i1, sem_g0, sem_g1, sem_w0, sem_w1):
        wid = lax.axis_index("s") * NC + lax.axis_index("c")
        nloop = (NCHUNK + NW - 1) // NW
        bufs = ((buf_a0, buf_b0, sem_i0, sem_g0, sem_w0),
                (buf_a1, buf_b1, sem_i1, sem_g1, sem_w1))

        def cid(j):
            return j * NW + wid

        def issue_idx(j, p):
            base = cid(j) * CH
            sem_i = bufs[p][2]
            pltpu.async_copy(si_hbm.at[pl.ds(base, CH)], si_v.at[p], sem_i)
            pltpu.async_copy(oi_hbm.at[pl.ds(base, CH)], oi_v.at[p], sem_i)

        def wait_idx(j, p):
            base = cid(j) * CH
            sem_i = bufs[p][2]
            pltpu.make_async_copy(si_hbm.at[pl.ds(base, CH)], si_v.at[p],
                                  sem_i).wait()
            pltpu.make_async_copy(oi_hbm.at[pl.ds(base, CH)], oi_v.at[p],
                                  sem_i).wait()

        @pl.when(cid(0) < NCHUNK)
        def _():
            issue_idx(0, 0)

        @pl.loop(0, nloop)
        def _(j):
            p = (j % 2).astype(jnp.int32) if hasattr(j % 2, "astype") else j % 2
            for par in range(2):
                @pl.when(((j % 2) == par) & (cid(j) < NCHUNK))
                def _(par=par):
                    buf_a, buf_b, sem_i, sem_g, sem_w = bufs[par]
                    wait_idx(j, par)

                    @pl.when(j >= 2)
                    def _():
                        pltpu.make_async_copy(
                            buf_a, ga_hbm.at[pl.ds(cid(j - 2) * CH, CH)],
                            sem_w).wait()
                        pltpu.make_async_copy(
                            buf_b, gb_hbm.at[pl.ds(cid(j - 2) * CH, CH)],
                            sem_w).wait()

                    pltpu.async_copy(a_hbm.at[si_v.at[par]], buf_a, sem_g)
                    pltpu.async_copy(b_hbm.at[oi_v.at[par]], buf_b, sem_g)

                    @pl.when(cid(j + 1) < NCHUNK)
                    def _():
                        issue_idx(j + 1, 1 - par)

                    pltpu.make_async_copy(a_hbm.at[si_v.at[par]], buf_a,
                                          sem_g).wait()
                    pltpu.make_async_copy(b_hbm.at[oi_v.at[par]], buf_b,
                                          sem_g).wait()
                    base = cid(j) * CH
                    pltpu.async_copy(buf_a, ga_hbm.at[pl.ds(base, CH)],
                                     sem_w)
                    pltpu.async_copy(buf_b, gb_hbm.at[pl.ds(base, CH)],
                                     sem_w)

        for par in range(2):
            jlast = nloop - 1 - ((nloop - 1 + par) % 2)

            @pl.when(cid(jlast) < NCHUNK)
            def _(par=par, jlast=jlast):
                buf_a, buf_b, _, _, sem_w = bufs[par]
                pltpu.make_async_copy(
                    buf_a, ga_hbm.at[pl.ds(cid(jlast) * CH, CH)],
                    sem_w).wait()
                pltpu.make_async_copy(
                    buf_b, gb_hbm.at[pl.ds(cid(jlast) * CH, CH)],
                    sem_w).wait()

    return k(a, b, s_idx, o_idx)


def _pass1_body(s_ref, ga_ref, gb_ref, pred_ref, w1p_ref, b1_ref,
                cw_ref, cb_ref, patch_ref, x2_ref, acc_ref):
    i = pl.program_id(0)
    h = (ga_ref[...] + gb_ref[...]
         + jnp.dot(pred_ref[...], w1p_ref[...], preferred_element_type=F32)
         + b1_ref[...])
    h = _leaky(h)
    g, e = _gmat(), _emat()
    x2 = _normcaps(h, g, e)
    rows = i * TILE + lax.broadcasted_iota(jnp.int32, (TILE, 1), 0)
    x2 = jnp.where(rows == s_ref[2], patch_ref[0:1, :], x2)
    x2 = jnp.where(rows == s_ref[3], patch_ref[1:2, :], x2)
    x2_ref[...] = x2
    y = jnp.dot(x2, cw_ref[...], preferred_element_type=F32) + cb_ref[...]

    @pl.when(i == 0)
    def _():
        acc_ref[...] = jnp.zeros((8, D1_OUT), F32)

    acc_ref[0:1, :] += jnp.sum(y, axis=0, keepdims=True)
    acc_ref[1:2, :] += jnp.sum(y * y, axis=0, keepdims=True)


def _pass1_call(ints, ga, gb, pred_vecs, w1, b1r, cw1, cb1r, patch):
    grid_spec = pltpu.PrefetchScalarGridSpec(
        num_scalar_prefetch=1,
        grid=(NT,),
        in_specs=[
            pl.BlockSpec((TILE, DREP), lambda i, s: (i, 0)),
            pl.BlockSpec((TILE, DREP), lambda i, s: (i, 0)),
            pl.BlockSpec((TILE, NHID), lambda i, s: (i, 0)),
            pl.BlockSpec((NHID, DREP), lambda i, s: (8, 0)),
            pl.BlockSpec((1, DREP), lambda i, s: (0, 0)),
            pl.BlockSpec((DREP, D1_OUT), lambda i, s: (0, 0)),
            pl.BlockSpec((1, D1_OUT), lambda i, s: (0, 0)),
            pl.BlockSpec((8, DREP), lambda i, s: (0, 0)),
        ],
        out_specs=[
            pl.BlockSpec((TILE, DREP), lambda i, s: (i, 0)),
            pl.BlockSpec((8, D1_OUT), lambda i, s: (0, 0)),
        ],
    )
    return pl.pallas_call(
        _pass1_body,
        grid_spec=grid_spec,
        out_shape=[
            jax.ShapeDtypeStruct((T, DREP), F32),
            jax.ShapeDtypeStruct((8, D1_OUT), F32),
        ],
    )(ints, ga, gb, pred_vecs, w1, b1r, cw1, cb1r, patch)


def _pass2_body(x2_ref, acc_ref, cw_ref, cb_ref, g_ref, b_ref,
                outs_ref, outp_ref, outo_ref):
    y = jnp.dot(x2_ref[...], cw_ref[...], preferred_element_type=F32) \
        + cb_ref[...]
    mu = acc_ref[0:1, :] * (1.0 / T)
    ey2 = acc_ref[1:2, :] * (1.0 / T)
    var = ey2 - mu * mu
    inv = 1.0 / jnp.sqrt(var + 1e-5)
    ob = _leaky((y - mu) * inv * g_ref[...] + b_ref[...])
    outs_ref[...] = ob[:, 0:H]
    outp_ref[...] = ob[:, H:H + DOUT]
    outo_ref[...] = ob[:, H + DOUT:]


def _pass2_call(x2, acc, cw1, cb1r, g1r, b1r):
    return pl.pallas_call(
        _pass2_body,
        grid=(NT,),
        in_specs=[
            pl.BlockSpec((TILE, DREP), lambda i: (i, 0)),
            pl.BlockSpec((8, D1_OUT), lambda i: (0, 0)),
            pl.BlockSpec((DREP, D1_OUT), lambda i: (0, 0)),
            pl.BlockSpec((1, D1_OUT), lambda i: (0, 0)),
            pl.BlockSpec((1, D1_OUT), lambda i: (0, 0)),
            pl.BlockSpec((1, D1_OUT), lambda i: (0, 0)),
        ],
        out_specs=[
            pl.BlockSpec((TILE, H), lambda i: (i, 0)),
            pl.BlockSpec((TILE, DOUT), lambda i: (i, 0)),
            pl.BlockSpec((TILE, H), lambda i: (i, 0)),
        ],
        out_shape=[
            jax.ShapeDtypeStruct((T, H), F32),
            jax.ShapeDtypeStruct((T, DOUT), F32),
            jax.ShapeDtypeStruct((T, H), F32),
        ],
    )(x2, acc, cw1, cb1r, g1r, b1r)


STRIPE = 624
TAIL = O - NS * STRIPE


def _sc_scatter(outs, outo, s_idx, o_idx, zrow):
    mesh = plsc.VectorSubcoreMesh(core_axis_name="c", subcore_axis_name="s")

    @functools.partial(
        pl.kernel,
        mesh=mesh,
        out_type=jax.ShapeDtypeStruct((NC * O, H), F32),
        scratch_types=[
            pltpu.VMEM((2, CHS), jnp.int32),
            pltpu.VMEM((2, CHS), jnp.int32),
            pltpu.VMEM((CHS, H), F32),
            pltpu.VMEM((CHS, H), F32),
            pltpu.VMEM((CHS, H), F32),
            pltpu.VMEM((CHS, H), F32),
            pltpu.VMEM_SHARED((O, H), F32),
            pltpu.SemaphoreType.DMA,
            pltpu.SemaphoreType.DMA,
        ],
    )
    def k(outs_hbm, outo_hbm, si_hbm, oi_hbm, zrow_hbm,
          pp_hbm, si_v, oi_v, vs0, vs1, vo0, vo1, pool_sh,
          sem_l0, sem_l1):
        cid_core = lax.axis_index("c")
        sid = lax.axis_index("s")
        wid = sid * NC + cid_core
        bufs = ((vs0, vo0, sem_l0), (vs1, vo1, sem_l1))

        pltpu.sync_copy(zrow_hbm, vs0)
        for t in range(SCHN + 1):
            sz = CHS if t < SCHN else SREM
            off = sid * STRIPE + t * CHS
            pltpu.sync_copy(vs0.at[pl.ds(0, sz)],
                            pool_sh.at[pl.ds(off, sz)])

        @pl.when(sid == 0)
        def _():
            pltpu.sync_copy(vs0.at[pl.ds(0, TAIL)],
                            pool_sh.at[pl.ds(NS * STRIPE, TAIL)])

        plsc.subcore_barrier()
        nloop = (NCHUNKS + NW - 1) // NW

        def cid(j):
            return j * NW + wid

        def issue_loads(j, p):
            base = cid(j) * CHS
            vs, vo, sem_l = bufs[p]
            pltpu.async_copy(si_hbm.at[pl.ds(base, CHS)], si_v.at[p], sem_l)
            pltpu.async_copy(outs_hbm.at[pl.ds(base, CHS)], vs, sem_l)
            pltpu.async_copy(oi_hbm.at[pl.ds(base, CHS)], oi_v.at[p], sem_l)
            pltpu.async_copy(outo_hbm.at[pl.ds(base, CHS)], vo, sem_l)

        def wait_loads(j, p):
            base = cid(j) * CHS
            vs, vo, sem_l = bufs[p]
            pltpu.make_async_copy(si_hbm.at[pl.ds(base, CHS)], si_v.at[p],
                                  sem_l).wait()
            pltpu.make_async_copy(outs_hbm.at[pl.ds(base, CHS)], vs,
                                  sem_l).wait()
            pltpu.make_async_copy(oi_hbm.at[pl.ds(base, CHS)], oi_v.at[p],
                                  sem_l).wait()
            pltpu.make_async_copy(outo_hbm.at[pl.ds(base, CHS)], vo,
                                  sem_l).wait()

        @pl.when(cid(0) < NCHUNKS)
        def _():
            issue_loads(0, 0)

        @pl.loop(0, nloop)
        def _(j):
            for par in range(2):
                @pl.when(((j % 2) == par) & (cid(j) < NCHUNKS))
                def _(par=par):
                    vs, vo, _ = bufs[par]
                    wait_loads(j, par)

                    @pl.when(cid(j + 1) < NCHUNKS)
                    def _():
                        issue_loads(j + 1, 1 - par)

                    pltpu.sync_copy(vs, pool_sh.at[si_v.at[par]], add=True)
                    pltpu.sync_copy(vo, pool_sh.at[oi_v.at[par]], add=True)

        plsc.subcore_barrier()

        for t in range(SCHN + 1):
            sz = CHS if t < SCHN else SREM
            soff = sid * STRIPE + t * CHS
            doff = cid_core * O + soff
            pltpu.sync_copy(pool_sh.at[pl.ds(soff, sz)],
                            vs0.at[pl.ds(0, sz)])
            pltpu.sync_copy(vs0.at[pl.ds(0, sz)],
                            pp_hbm.at[pl.ds(doff, sz)])

        @pl.when(sid == 0)
        def _():
            tbase = cid_core * O + NS * STRIPE
            pltpu.sync_copy(pool_sh.at[pl.ds(NS * STRIPE, TAIL)],
                            vo0.at[pl.ds(0, TAIL)])
            pltpu.sync_copy(vo0.at[pl.ds(0, TAIL)],
                            pp_hbm.at[pl.ds(tbase, TAIL)])

    return k(outs, outo, s_idx, o_idx, zrow)


def _sc_counts(s_idx, o_idx, zrow, ones):
    mesh = plsc.VectorSubcoreMesh(core_axis_name="c", subcore_axis_name="s")

    @functools.partial(
        pl.kernel,
        mesh=mesh,
        out_type=jax.ShapeDtypeStruct((NC * O, H), F32),
        scratch_types=[
            pltpu.VMEM((2, CHS), jnp.int32),
            pltpu.VMEM((2, CHS), jnp.int32),
            pltpu.VMEM((CHS, H), F32),
            pltpu.VMEM((CHS, H), F32),
            pltpu.VMEM_SHARED((O, H), F32),
            pltpu.SemaphoreType.DMA,
            pltpu.SemaphoreType.DMA,
        ],
    )
    def k(si_hbm, oi_hbm, zrow_hbm, ones_hbm, cc_hbm,
          si_v, oi_v, val_v, ones_v, cnt_sh, sem_l0, sem_l1):
        cid_core = lax.axis_index("c")
        sid = lax.axis_index("s")
        wid = sid * NC + cid_core
        sems = (sem_l0, sem_l1)

        pltpu.sync_copy(zrow_hbm, val_v)
        pltpu.sync_copy(ones_hbm, ones_v)
        for t in range(SCHN + 1):
            sz = CHS if t < SCHN else SREM
            off = sid * STRIPE + t * CHS
            pltpu.sync_copy(val_v.at[pl.ds(0, sz)],
                            cnt_sh.at[pl.ds(off, sz)])

        @pl.when(sid == 0)
        def _():
            pltpu.sync_copy(val_v.at[pl.ds(0, TAIL)],
                            cnt_sh.at[pl.ds(NS * STRIPE, TAIL)])

        plsc.subcore_barrier()
        nloop = (NCHUNKS + NW - 1) // NW

        def cid(j):
            return j * NW + wid

        def issue_idx(j, p):
            base = cid(j) * CHS
            pltpu.async_copy(si_hbm.at[pl.ds(base, CHS)], si_v.at[p],
                             sems[p])
            pltpu.async_copy(oi_hbm.at[pl.ds(base, CHS)], oi_v.at[p],
                             sems[p])

        def wait_idx(j, p):
            base = cid(j) * CHS
            pltpu.make_async_copy(si_hbm.at[pl.ds(base, CHS)], si_v.at[p],
                                  sems[p]).wait()
            pltpu.make_async_copy(oi_hbm.at[pl.ds(base, CHS)], oi_v.at[p],
                                  sems[p]).wait()

        @pl.when(cid(0) < NCHUNKS)
        def _():
            issue_idx(0, 0)

        @pl.loop(0, nloop)
        def _(j):
            for par in range(2):
                @pl.when(((j % 2) == par) & (cid(j) < NCHUNKS))
                def _(par=par):
                    wait_idx(j, par)

                    @pl.when(cid(j + 1) < NCHUNKS)
                    def _():
                        issue_idx(j + 1, 1 - par)

                    pltpu.sync_copy(ones_v, cnt_sh.at[si_v.at[par]],
                                    add=True)
                    pltpu.sync_copy(ones_v, cnt_sh.at[oi_v.at[par]],
                                    add=True)

        plsc.subcore_barrier()

        for t in range(SCHN + 1):
            sz = CHS if t < SCHN else SREM
            soff = sid * STRIPE + t * CHS
            doff = cid_core * O + soff
            pltpu.sync_copy(cnt_sh.at[pl.ds(soff, sz)],
                            val_v.at[pl.ds(0, sz)])
            pltpu.sync_copy(val_v.at[pl.ds(0, sz)],
                            cc_hbm.at[pl.ds(doff, sz)])

        @pl.when(sid == 0)
        def _():
            tbase = cid_core * O + NS * STRIPE
            pltpu.sync_copy(cnt_sh.at[pl.ds(NS * STRIPE, TAIL)],
                            val_v.at[pl.ds(0, TAIL)])
            pltpu.sync_copy(val_v.at[pl.ds(0, TAIL)],
                            cc_hbm.at[pl.ds(tbase, TAIL)])

    return k(s_idx, o_idx, zrow, ones)


def _final_body(s_ref, pp_ref, cc_ref, pw_ref, pb_ref, cw_ref, cb_ref,
                g2_ref, b2_ref, out_ref, scr_ref):
    pooled = pp_ref[0:O, :] + pp_ref[O:2 * O, :]
    cnt = cc_ref[0:O, 0:1] + cc_ref[O:2 * O, 0:1]
    c0 = jnp.maximum(cnt, 1.0)
    pavg = pooled / c0
    h = _leaky(jnp.dot(pavg, pw_ref[...], preferred_element_type=F32)
               + pb_ref[...])
    scr_ref[...] = h
    g, e = _gmat(), _emat()
    vs = [scr_ref[pl.ds(s_ref[k], 1), :] for k in range(4)]
    p0, p1 = _routing_patch(vs[0], vs[1], vs[2], vs[3],
                            s_ref[0], s_ref[1], s_ref[2], s_ref[3], g, e)
    x2 = _normcaps(h, g, e)
    scr_ref[...] = x2
    scr_ref[pl.ds(s_ref[2], 1), :] = p0
    scr_ref[pl.ds(s_ref[3], 1), :] = p1
    y = jnp.dot(scr_ref[...], cw_ref[...], preferred_element_type=F32) \
        + cb_ref[...]
    mu = jnp.mean(y, axis=0, keepdims=True)
    xc = y - mu
    var = jnp.mean(xc * xc, axis=0, keepdims=True)
    out_ref[...] = _leaky(xc * (1.0 / jnp.sqrt(var + 1e-5)) * g2_ref[...]
                          + b2_ref[...])


def _final_call(ints, pp, cc, pw2, pb2r, cw2, cb2r, g2r, b2r):
    grid_spec = pltpu.PrefetchScalarGridSpec(
        num_scalar_prefetch=1,
        grid=(1,),
        in_specs=[
            pl.BlockSpec((NC * O, H), lambda i, s: (0, 0)),
            pl.BlockSpec((NC * O, H), lambda i, s: (0, 0)),
            pl.BlockSpec((DREP, DREP), lambda i, s: (0, 0)),
            pl.BlockSpec((1, DREP), lambda i, s: (0, 0)),
            pl.BlockSpec((DREP, DOUT), lambda i, s: (0, 0)),
            pl.BlockSpec((1, DOUT), lambda i, s: (0, 0)),
            pl.BlockSpec((1, DOUT), lambda i, s: (0, 0)),
            pl.BlockSpec((1, DOUT), lambda i, s: (0, 0)),
        ],
        out_specs=pl.BlockSpec((O, DOUT), lambda i, s: (0, 0)),
        scratch_shapes=[pltpu.VMEM((O, DREP), F32)],
    )
    return pl.pallas_call(
        _final_body,
        grid_spec=grid_spec,
        out_shape=jax.ShapeDtypeStruct((O, DOUT), F32),
    )(ints, pp, cc, pw2, pb2r, cw2, cb2r, g2r, b2r)


def kernel(obj_vecs, pred_vecs, edges, pca_W1, pca_b1, clf_W1, clf_b1,
           bn1_gamma, bn1_beta, pca_W2, pca_b2, clf_W2, clf_b2,
           bn2_gamma, bn2_beta):
    s_idx = edges[:, 0]
    o_idx = edges[:, 1]
    idx4 = jnp.stack([edges[0, 0], edges[0, 1], edges[1, 0], edges[1, 1]])
    rows4 = edges[idx4]
    ints = jnp.concatenate([idx4, rows4.reshape(-1),
                            jnp.zeros((4,), jnp.int32)])
    pred4 = jnp.concatenate([pred_vecs[idx4], jnp.zeros((4, NHID), F32)], 0)

    zrow = jnp.zeros((CHS, H), F32)
    onesr = jnp.ones((CHS, H), F32)
    cc = _sc_counts(s_idx, o_idx, zrow, onesr)
    a, b, patch = _prep_call(ints, obj_vecs, pca_W1,
                             pca_b1.reshape(1, -1), pred4)
    ga, gb = _sc_gather(a, b, s_idx, o_idx)
    x2, acc = _pass1_call(ints, ga, gb, pred_vecs, pca_W1,
                          pca_b1.reshape(1, -1), clf_W1,
                          clf_b1.reshape(1, -1), patch)
    outs, outp, outo = _pass2_call(x2, acc, clf_W1, clf_b1.reshape(1, -1),
                                   bn1_gamma.reshape(1, -1),
                                   bn1_beta.reshape(1, -1))
    pp = _sc_scatter(outs, outo, s_idx, o_idx, zrow)
    obj_out = _final_call(ints, pp, cc, pca_W2, pca_b2.reshape(1, -1),
                          clf_W2, clf_b2.reshape(1, -1),
                          bn2_gamma.reshape(1, -1), bn2_beta.reshape(1, -1))
    return (obj_out, outp)


# speedup vs baseline: 1.2878x; 1.0071x over previous
import functools

import jax
import jax.numpy as jnp
from jax import lax
from jax.experimental import pallas as pl
from jax.experimental.pallas import tpu as pltpu
from jax.experimental.pallas import tpu_sc as plsc

O = 10000
T = 160000
H = 128
DOUT = 32
NCAPS = 8
NHID = 16
DREP = NCAPS * NHID
D1_OUT = 2 * H + DOUT
NLAYER = 2
ROUTIT = 3

TILE = 8000
NT = T // TILE

NC, NS = 2, 16
NW = NC * NS
CH = 128
NCHUNK = T // CH
HEXT = H + NHID
CHS = 64
NCHUNKS = T // CHS
SCHN = 624 // CHS
SREM = 624 - SCHN * CHS
F32 = jnp.float32


def _leaky(x):
    return jnp.where(x >= 0, x, 0.01 * x)


def _gmat():
    r = lax.broadcasted_iota(jnp.int32, (DREP, NCAPS), 0) // NHID
    c = lax.broadcasted_iota(jnp.int32, (DREP, NCAPS), 1)
    return (r == c).astype(F32)


def _emat():
    r = lax.broadcasted_iota(jnp.int32, (NCAPS, DREP), 0)
    c = lax.broadcasted_iota(jnp.int32, (NCAPS, DREP), 1) // NHID
    return (r == c).astype(F32)


def _hdot(a, b):
    return jnp.dot(a, b, preferred_element_type=F32,
                   precision=lax.Precision.HIGHEST)


def _normcaps(x, g, e):
    n2 = _hdot(x * x, g)
    inv = 1.0 / jnp.maximum(jnp.sqrt(n2), 1e-12)
    return x * _hdot(inv, e)


def _softmax8(p):
    m = jnp.max(p, axis=1, keepdims=True)
    ex = jnp.exp(p - m)
    return ex / jnp.sum(ex, axis=1, keepdims=True)


def _routing_patch(v0, v1, v2, v3, a0, a1, b0, b1, g, e):
    v = [v0, v1, v2, v3]
    beq = b0 == b1
    for _ in range(NLAYER):
        w = [_normcaps(vk, g, e) for vk in v]
        z0, z1 = w[0], w[1]
        ub0, ub1 = w[2], w[3]
        for _ in range(ROUTIT):
            p0 = _softmax8(_hdot(z0 * ub0, g))
            s0 = z0 * _hdot(p0, e)
            p1 = _softmax8(_hdot(z1 * ub1, g))
            s1 = z1 * _hdot(p1, e)
            n_same = _normcaps(w[2] + s0 + s1, g, e)
            n_b0 = _normcaps(w[2] + s0, g, e)
            n_b1 = _normcaps(w[3] + s1, g, e)
            ub0 = jnp.where(beq, n_same, n_b0)
            ub1 = jnp.where(beq, n_same, n_b1)
        v = [
            jnp.where(a0 == b0, ub0, jnp.where(a0 == b1, ub1, z0)),
            jnp.where(a1 == b0, ub0, jnp.where(a1 == b1, ub1, z1)),
            ub0,
            ub1,
        ]
    return v[2], v[3]


def _prep_body(s_ref, obj_ref, w1_ref, b1_ref, pred4_ref,
               a_ref, b_ref, patch_ref):
    obj = obj_ref[...]
    a_ref[...] = jnp.dot(obj, w1_ref[0:DREP, :], preferred_element_type=F32)
    b_ref[...] = jnp.dot(obj, w1_ref[DREP + NHID:, :],
                         preferred_element_type=F32)
    g, e = _gmat(), _emat()
    wp = w1_ref[DREP:DREP + NHID, :]
    vs = []
    for k in range(4):
        sr = s_ref[4 + 2 * k]
        orr = s_ref[5 + 2 * k]
        hrow = (a_ref[pl.ds(sr, 1), :] + b_ref[pl.ds(orr, 1), :]
                + jnp.dot(pred4_ref[k:k + 1, :], wp,
                          preferred_element_type=F32)
                + b1_ref[...])
        vs.append(_leaky(hrow))
    p0, p1 = _routing_patch(vs[0], vs[1], vs[2], vs[3],
                            s_ref[0], s_ref[1], s_ref[2], s_ref[3], g, e)
    patch_ref[...] = jnp.concatenate([p0, p1, jnp.zeros((6, DREP), F32)], 0)


def _prep_call(ints, obj_vecs, w1, b1r, pred4):
    grid_spec = pltpu.PrefetchScalarGridSpec(
        num_scalar_prefetch=1,
        grid=(1,),
        in_specs=[
            pl.BlockSpec((O, DREP), lambda i, s: (0, 0)),
            pl.BlockSpec((2 * DREP + NHID, DREP), lambda i, s: (0, 0)),
            pl.BlockSpec((1, DREP), lambda i, s: (0, 0)),
            pl.BlockSpec((8, NHID), lambda i, s: (0, 0)),
        ],
        out_specs=[
            pl.BlockSpec((O, DREP), lambda i, s: (0, 0)),
            pl.BlockSpec((O, DREP), lambda i, s: (0, 0)),
            pl.BlockSpec((8, DREP), lambda i, s: (0, 0)),
        ],
    )
    return pl.pallas_call(
        _prep_body,
        grid_spec=grid_spec,
        out_shape=[
            jax.ShapeDtypeStruct((O, DREP), F32),
            jax.ShapeDtypeStruct((O, DREP), F32),
            jax.ShapeDtypeStruct((8, DREP), F32),
        ],
    )(ints, obj_vecs, w1, b1r, pred4)


def _sc_gather(a, b, s_idx, o_idx):
    mesh = plsc.VectorSubcoreMesh(core_axis_name="c", subcore_axis_name="s")

    @functools.partial(
        pl.kernel,
        mesh=mesh,
        out_type=[
            jax.ShapeDtypeStruct((T, DREP), F32),
            jax.ShapeDtypeStruct((T, DREP), F32),
        ],
        scratch_types=[
            pltpu.VMEM((2, CH), jnp.int32),
            pltpu.VMEM((2, CH), jnp.int32),
            pltpu.VMEM((CH, DREP), F32),
            pltpu.VMEM((CH, DREP), F32),
            pltpu.VMEM((CH, DREP), F32),
            pltpu.VMEM((CH, DREP), F32),
            pltpu.SemaphoreType.DMA,
            pltpu.SemaphoreType.DMA,
            pltpu.SemaphoreType.DMA,
            pltpu.SemaphoreType.DMA,
            pltpu.SemaphoreType.DMA,
            pltpu.SemaphoreType.DMA,
        ],
    )
    def k(a_hbm, b_hbm, si_hbm, oi_hbm, ga_hbm, gb_hbm,
          si_v, oi_v, buf_a0, buf_a1, buf_b0, buf_b1,
          sem_i0, sem_i1, sem_g0, sem_g1, sem_w0, sem_w1):
        wid = lax.axis_index("s") * NC + lax.axis_index("c")
        nloop = (NCHUNK + NW - 1) // NW
        bufs = ((buf_a0, buf_b0, sem_i0, sem_g0, sem_w0),
                (buf_a1, buf_b1, sem_i1, sem_g1, sem_w1))

        def cid(j):
            return j * NW + wid

        def issue_idx(j, p):
            base = cid(j) * CH
            sem_i = bufs[p][2]
            pltpu.async_copy(si_hbm.at[pl.ds(base, CH)], si_v.at[p], sem_i)
            pltpu.async_copy(oi_hbm.at[pl.ds(base, CH)], oi_v.at[p], sem_i)

        def wait_idx(j, p):
            base = cid(j) * CH
            sem_i = bufs[p][2]
            pltpu.make_async_copy(si_hbm.at[pl.ds(base, CH)], si_v.at[p],
                                  sem_i).wait()
            pltpu.make_async_copy(oi_hbm.at[pl.ds(base, CH)], oi_v.at[p],
                                  sem_i).wait()

        @pl.when(cid(0) < NCHUNK)
        def _():
            issue_idx(0, 0)

        @pl.loop(0, nloop)
        def _(j):
            p = (j % 2).astype(jnp.int32) if hasattr(j % 2, "astype") else j % 2
            for par in range(2):
                @pl.when(((j % 2) == par) & (cid(j) < NCHUNK))
                def _(par=par):
                    buf_a, buf_b, sem_i, sem_g, sem_w = bufs[par]
                    wait_idx(j, par)

                    @pl.when(j >= 2)
                    def _():
                        pltpu.make_async_copy(
                            buf_a, ga_hbm.at[pl.ds(cid(j - 2) * CH, CH)],
                            sem_w).wait()
                        pltpu.make_async_copy(
                            buf_b, gb_hbm.at[pl.ds(cid(j - 2) * CH, CH)],
                            sem_w).wait()

                    pltpu.async_copy(a_hbm.at[si_v.at[par]], buf_a, sem_g)
                    pltpu.async_copy(b_hbm.at[oi_v.at[par]], buf_b, sem_g)

                    @pl.when(cid(j + 1) < NCHUNK)
                    def _():
                        issue_idx(j + 1, 1 - par)

                    pltpu.make_async_copy(a_hbm.at[si_v.at[par]], buf_a,
                                          sem_g).wait()
                    pltpu.make_async_copy(b_hbm.at[oi_v.at[par]], buf_b,
                                          sem_g).wait()
                    base = cid(j) * CH
                    pltpu.async_copy(buf_a, ga_hbm.at[pl.ds(base, CH)],
                                     sem_w)
                    pltpu.async_copy(buf_b, gb_hbm.at[pl.ds(base, CH)],
                                     sem_w)

        for par in range(2):
            jlast = nloop - 1 - ((nloop - 1 + par) % 2)

            @pl.when(cid(jlast) < NCHUNK)
            def _(par=par, jlast=jlast):
                buf_a, buf_b, _, _, sem_w = bufs[par]
                pltpu.make_async_copy(
                    buf_a, ga_hbm.at[pl.ds(cid(jlast) * CH, CH)],
                    sem_w).wait()
                pltpu.make_async_copy(
                    buf_b, gb_hbm.at[pl.ds(cid(jlast) * CH, CH)],
                    sem_w).wait()

    return k(a, b, s_idx, o_idx)


def _pass1_body(s_ref, ga_ref, gb_ref, pred_ref, w1p_ref, b1_ref,
                cw_ref, cb_ref, patch_ref, x2_ref, acc_ref):
    i = pl.program_id(0)
    h = (ga_ref[...] + gb_ref[...]
         + jnp.dot(pred_ref[...], w1p_ref[...], preferred_element_type=F32)
         + b1_ref[...])
    h = _leaky(h)
    g, e = _gmat(), _emat()
    x2 = _normcaps(h, g, e)
    rows = i * TILE + lax.broadcasted_iota(jnp.int32, (TILE, 1), 0)
    x2 = jnp.where(rows == s_ref[2], patch_ref[0:1, :], x2)
    x2 = jnp.where(rows == s_ref[3], patch_ref[1:2, :], x2)
    x2_ref[...] = x2
    y = jnp.dot(x2, cw_ref[...], preferred_element_type=F32) + cb_ref[...]

    @pl.when(i == 0)
    def _():
        acc_ref[...] = jnp.zeros((8, D1_OUT), F32)

    acc_ref[0:1, :] += jnp.sum(y, axis=0, keepdims=True)
    acc_ref[1:2, :] += jnp.sum(y * y, axis=0, keepdims=True)


def _pass1_call(ints, ga, gb, pred_vecs, w1, b1r, cw1, cb1r, patch):
    grid_spec = pltpu.PrefetchScalarGridSpec(
        num_scalar_prefetch=1,
        grid=(NT,),
        in_specs=[
            pl.BlockSpec((TILE, DREP), lambda i, s: (i, 0)),
            pl.BlockSpec((TILE, DREP), lambda i, s: (i, 0)),
            pl.BlockSpec((TILE, NHID), lambda i, s: (i, 0)),
            pl.BlockSpec((NHID, DREP), lambda i, s: (8, 0)),
            pl.BlockSpec((1, DREP), lambda i, s: (0, 0)),
            pl.BlockSpec((DREP, D1_OUT), lambda i, s: (0, 0)),
            pl.BlockSpec((1, D1_OUT), lambda i, s: (0, 0)),
            pl.BlockSpec((8, DREP), lambda i, s: (0, 0)),
        ],
        out_specs=[
            pl.BlockSpec((TILE, DREP), lambda i, s: (i, 0)),
            pl.BlockSpec((8, D1_OUT), lambda i, s: (0, 0)),
        ],
    )
    return pl.pallas_call(
        _pass1_body,
        grid_spec=grid_spec,
        out_shape=[
            jax.ShapeDtypeStruct((T, DREP), F32),
            jax.ShapeDtypeStruct((8, D1_OUT), F32),
        ],
    )(ints, ga, gb, pred_vecs, w1, b1r, cw1, cb1r, patch)


def _pass2_body(x2_ref, acc_ref, cw_ref, cb_ref, g_ref, b_ref,
                outs_ref, outp_ref, outo_ref):
    y = jnp.dot(x2_ref[...], cw_ref[...], preferred_element_type=F32) \
        + cb_ref[...]
    mu = acc_ref[0:1, :] * (1.0 / T)
    ey2 = acc_ref[1:2, :] * (1.0 / T)
    var = ey2 - mu * mu
    inv = 1.0 / jnp.sqrt(var + 1e-5)
    ob = _leaky((y - mu) * inv * g_ref[...] + b_ref[...])
    outs_ref[...] = ob[:, 0:H]
    outp_ref[...] = ob[:, H:H + DOUT]
    outo_ref[...] = ob[:, H + DOUT:]


def _pass2_call(x2, acc, cw1, cb1r, g1r, b1r):
    return pl.pallas_call(
        _pass2_body,
        grid=(NT,),
        in_specs=[
            pl.BlockSpec((TILE, DREP), lambda i: (i, 0)),
            pl.BlockSpec((8, D1_OUT), lambda i: (0, 0)),
            pl.BlockSpec((DREP, D1_OUT), lambda i: (0, 0)),
            pl.BlockSpec((1, D1_OUT), lambda i: (0, 0)),
            pl.BlockSpec((1, D1_OUT), lambda i: (0, 0)),
            pl.BlockSpec((1, D1_OUT), lambda i: (0, 0)),
        ],
        out_specs=[
            pl.BlockSpec((TILE, H), lambda i: (i, 0)),
            pl.BlockSpec((TILE, DOUT), lambda i: (i, 0)),
            pl.BlockSpec((TILE, H), lambda i: (i, 0)),
        ],
        out_shape=[
            jax.ShapeDtypeStruct((T, H), F32),
            jax.ShapeDtypeStruct((T, DOUT), F32),
            jax.ShapeDtypeStruct((T, H), F32),
        ],
    )(x2, acc, cw1, cb1r, g1r, b1r)


STRIPE = 624
TAIL = O - NS * STRIPE


def _sc_scatter(outs, outo, s_idx, o_idx, zrow):
    mesh = plsc.VectorSubcoreMesh(core_axis_name="c", subcore_axis_name="s")

    @functools.partial(
        pl.kernel,
        mesh=mesh,
        out_type=jax.ShapeDtypeStruct((NC * O, H), F32),
        scratch_types=[
            pltpu.VMEM((2, CHS), jnp.int32),
            pltpu.VMEM((2, CHS), jnp.int32),
            pltpu.VMEM((CHS, H), F32),
            pltpu.VMEM((CHS, H), F32),
            pltpu.VMEM((CHS, H), F32),
            pltpu.VMEM((CHS, H), F32),
            pltpu.VMEM_SHARED((O, H), F32),
            pltpu.SemaphoreType.DMA,
            pltpu.SemaphoreType.DMA,
        ],
    )
    def k(outs_hbm, outo_hbm, si_hbm, oi_hbm, zrow_hbm,
          pp_hbm, si_v, oi_v, vs0, vs1, vo0, vo1, pool_sh,
          sem_l0, sem_l1):
        cid_core = lax.axis_index("c")
        sid = lax.axis_index("s")
        wid = sid * NC + cid_core
        bufs = ((vs0, vo0, sem_l0), (vs1, vo1, sem_l1))

        pltpu.sync_copy(zrow_hbm, vs0)
        for t in range(SCHN + 1):
            sz = CHS if t < SCHN else SREM
            off = sid * STRIPE + t * CHS
            pltpu.sync_copy(vs0.at[pl.ds(0, sz)],
                            pool_sh.at[pl.ds(off, sz)])

        @pl.when(sid == 0)
        def _():
            pltpu.sync_copy(vs0.at[pl.ds(0, TAIL)],
                            pool_sh.at[pl.ds(NS * STRIPE, TAIL)])

        plsc.subcore_barrier()
        nloop = (NCHUNKS + NW - 1) // NW

        def cid(j):
            return j * NW + wid

        def issue_loads(j, p):
            base = cid(j) * CHS
            vs, vo, sem_l = bufs[p]
            pltpu.async_copy(si_hbm.at[pl.ds(base, CHS)], si_v.at[p], sem_l)
            pltpu.async_copy(outs_hbm.at[pl.ds(base, CHS)], vs, sem_l)
            pltpu.async_copy(oi_hbm.at[pl.ds(base, CHS)], oi_v.at[p], sem_l)
            pltpu.async_copy(outo_hbm.at[pl.ds(base, CHS)], vo, sem_l)

        def wait_loads(j, p):
            base = cid(j) * CHS
            vs, vo, sem_l = bufs[p]
            pltpu.make_async_copy(si_hbm.at[pl.ds(base, CHS)], si_v.at[p],
                                  sem_l).wait()
            pltpu.make_async_copy(outs_hbm.at[pl.ds(base, CHS)], vs,
                                  sem_l).wait()
            pltpu.make_async_copy(oi_hbm.at[pl.ds(base, CHS)], oi_v.at[p],
                                  sem_l).wait()
            pltpu.make_async_copy(outo_hbm.at[pl.ds(base, CHS)], vo,
                                  sem_l).wait()

        @pl.when(cid(0) < NCHUNKS)
        def _():
            issue_loads(0, 0)

        @pl.loop(0, nloop)
        def _(j):
            for par in range(2):
                @pl.when(((j % 2) == par) & (cid(j) < NCHUNKS))
                def _(par=par):
                    vs, vo, _ = bufs[par]
                    wait_loads(j, par)

                    @pl.when(cid(j + 1) < NCHUNKS)
                    def _():
                        issue_loads(j + 1, 1 - par)

                    pltpu.sync_copy(vs, pool_sh.at[si_v.at[par]], add=True)
                    pltpu.sync_copy(vo, pool_sh.at[oi_v.at[par]], add=True)

        plsc.subcore_barrier()

        for t in range(SCHN + 1):
            sz = CHS if t < SCHN else SREM
            soff = sid * STRIPE + t * CHS
            doff = cid_core * O + soff
            pltpu.sync_copy(pool_sh.at[pl.ds(soff, sz)],
                            vs0.at[pl.ds(0, sz)])
            pltpu.sync_copy(vs0.at[pl.ds(0, sz)],
                            pp_hbm.at[pl.ds(doff, sz)])

        @pl.when(sid == 0)
        def _():
            tbase = cid_core * O + NS * STRIPE
            pltpu.sync_copy(pool_sh.at[pl.ds(NS * STRIPE, TAIL)],
                            vo0.at[pl.ds(0, TAIL)])
            pltpu.sync_copy(vo0.at[pl.ds(0, TAIL)],
                            pp_hbm.at[pl.ds(tbase, TAIL)])

    return k(outs, outo, s_idx, o_idx, zrow)


def _sc_counts(s_idx, o_idx, zrow, ones):
    mesh = plsc.VectorSubcoreMesh(core_axis_name="c", subcore_axis_name="s")

    @functools.partial(
        pl.kernel,
        mesh=mesh,
        out_type=jax.ShapeDtypeStruct((NC * O, H), F32),
        scratch_types=[
            pltpu.VMEM((2, CHS), jnp.int32),
            pltpu.VMEM((2, CHS), jnp.int32),
            pltpu.VMEM((CHS, H), F32),
            pltpu.VMEM((CHS, H), F32),
            pltpu.VMEM_SHARED((O, H), F32),
            pltpu.SemaphoreType.DMA,
            pltpu.SemaphoreType.DMA,
        ],
    )
    def k(si_hbm, oi_hbm, zrow_hbm, ones_hbm, cc_hbm,
          si_v, oi_v, val_v, ones_v, cnt_sh, sem_l0, sem_l1):
        cid_core = lax.axis_index("c")
        sid = lax.axis_index("s")
        wid = sid * NC + cid_core
        sems = (sem_l0, sem_l1)

        pltpu.sync_copy(zrow_hbm, val_v)
        pltpu.sync_copy(ones_hbm, ones_v)
        for t in range(SCHN + 1):
            sz = CHS if t < SCHN else SREM
            off = sid * STRIPE + t * CHS
            pltpu.sync_copy(val_v.at[pl.ds(0, sz)],
                            cnt_sh.at[pl.ds(off, sz)])

        @pl.when(sid == 0)
        def _():
            pltpu.sync_copy(val_v.at[pl.ds(0, TAIL)],
                            cnt_sh.at[pl.ds(NS * STRIPE, TAIL)])

        plsc.subcore_barrier()
        nloop = (NCHUNKS + NW - 1) // NW

        def cid(j):
            return j * NW + wid

        def issue_idx(j, p):
            base = cid(j) * CHS
            pltpu.async_copy(si_hbm.at[pl.ds(base, CHS)], si_v.at[p],
                             sems[p])
            pltpu.async_copy(oi_hbm.at[pl.ds(base, CHS)], oi_v.at[p],
                             sems[p])

        def wait_idx(j, p):
            base = cid(j) * CHS
            pltpu.make_async_copy(si_hbm.at[pl.ds(base, CHS)], si_v.at[p],
                                  sems[p]).wait()
            pltpu.make_async_copy(oi_hbm.at[pl.ds(base, CHS)], oi_v.at[p],
                                  sems[p]).wait()

        @pl.when(cid(0) < NCHUNKS)
        def _():
            issue_idx(0, 0)

        @pl.loop(0, nloop)
        def _(j):
            for par in range(2):
                @pl.when(((j % 2) == par) & (cid(j) < NCHUNKS))
                def _(par=par):
                    wait_idx(j, par)

                    @pl.when(cid(j + 1) < NCHUNKS)
                    def _():
                        issue_idx(j + 1, 1 - par)

                    pltpu.sync_copy(ones_v, cnt_sh.at[si_v.at[par]],
                                    add=True)
                    pltpu.sync_copy(ones_v, cnt_sh.at[oi_v.at[par]],
                                    add=True)

        plsc.subcore_barrier()

        for t in range(SCHN + 1):
            sz = CHS if t < SCHN else SREM
            soff = sid * STRIPE + t * CHS
            doff = cid_core * O + soff
            pltpu.sync_copy(cnt_sh.at[pl.ds(soff, sz)],
                            val_v.at[pl.ds(0, sz)])
            pltpu.sync_copy(val_v.at[pl.ds(0, sz)],
                            cc_hbm.at[pl.ds(doff, sz)])

        @pl.when(sid == 0)
        def _():
            tbase = cid_core * O + NS * STRIPE
            pltpu.sync_copy(cnt_sh.at[pl.ds(NS * STRIPE, TAIL)],
                            val_v.at[pl.ds(0, TAIL)])
            pltpu.sync_copy(val_v.at[pl.ds(0, TAIL)],
                            cc_hbm.at[pl.ds(tbase, TAIL)])

    return k(s_idx, o_idx, zrow, ones)


def _final_body(s_ref, pp_ref, cc_ref, pw_ref, pb_ref, cw_ref, cb_ref,
                g2_ref, b2_ref, out_ref, scr_ref):
    pooled = pp_ref[0:O, :] + pp_ref[O:2 * O, :]
    cnt = cc_ref[0:O, 0:1] + cc_ref[O:2 * O, 0:1]
    c0 = jnp.maximum(cnt, 1.0)
    pavg = pooled / c0
    h = _leaky(jnp.dot(pavg, pw_ref[...], preferred_element_type=F32)
               + pb_ref[...])
    scr_ref[...] = h
    g, e = _gmat(), _emat()
    vs = [scr_ref[pl.ds(s_ref[k], 1), :] for k in range(4)]
    p0, p1 = _routing_patch(vs[0], vs[1], vs[2], vs[3],
                            s_ref[0], s_ref[1], s_ref[2], s_ref[3], g, e)
    x2 = _normcaps(h, g, e)
    scr_ref[...] = x2
    scr_ref[pl.ds(s_ref[2], 1), :] = p0
    scr_ref[pl.ds(s_ref[3], 1), :] = p1
    y = jnp.dot(scr_ref[...], cw_ref[...], preferred_element_type=F32) \
        + cb_ref[...]
    mu = jnp.mean(y, axis=0, keepdims=True)
    xc = y - mu
    var = jnp.mean(xc * xc, axis=0, keepdims=True)
    out_ref[...] = _leaky(xc * (1.0 / jnp.sqrt(var + 1e-5)) * g2_ref[...]
                          + b2_ref[...])


def _final_call(ints, pp, cc, pw2, pb2r, cw2, cb2r, g2r, b2r):
    grid_spec = pltpu.PrefetchScalarGridSpec(
        num_scalar_prefetch=1,
        grid=(1,),
        in_specs=[
            pl.BlockSpec((NC * O, H), lambda i, s: (0, 0)),
            pl.BlockSpec((NC * O, H), lambda i, s: (0, 0)),
            pl.BlockSpec((DREP, DREP), lambda i, s: (0, 0)),
            pl.BlockSpec((1, DREP), lambda i, s: (0, 0)),
            pl.BlockSpec((DREP, DOUT), lambda i, s: (0, 0)),
            pl.BlockSpec((1, DOUT), lambda i, s: (0, 0)),
            pl.BlockSpec((1, DOUT), lambda i, s: (0, 0)),
            pl.BlockSpec((1, DOUT), lambda i, s: (0, 0)),
        ],
        out_specs=pl.BlockSpec((O, DOUT), lambda i, s: (0, 0)),
        scratch_shapes=[pltpu.VMEM((O, DREP), F32)],
    )
    return pl.pallas_call(
        _final_body,
        grid_spec=grid_spec,
        out_shape=jax.ShapeDtypeStruct((O, DOUT), F32),
    )(ints, pp, cc, pw2, pb2r, cw2, cb2r, g2r, b2r)


def kernel(obj_vecs, pred_vecs, edges, pca_W1, pca_b1, clf_W1, clf_b1,
           bn1_gamma, bn1_beta, pca_W2, pca_b2, clf_W2, clf_b2,
           bn2_gamma, bn2_beta):
    s_idx = edges[:, 0]
    o_idx = edges[:, 1]
    idx4 = jnp.stack([edges[0, 0], edges[0, 1], edges[1, 0], edges[1, 1]])
    rows4 = edges[idx4]
    ints = jnp.concatenate([idx4, rows4.reshape(-1),
                            jnp.zeros((4,), jnp.int32)])
    pred4 = jnp.concatenate([pred_vecs[idx4], jnp.zeros((4, NHID), F32)], 0)

    zrow = jnp.zeros((CHS, H), F32)
    onesr = jnp.ones((CHS, H), F32)
    cc = _sc_counts(s_idx, o_idx, zrow, onesr)
    a, b, patch = _prep_call(ints, obj_vecs, pca_W1,
                             pca_b1.reshape(1, -1), pred4)
    ga, gb = _sc_gather(a, b, s_idx, o_idx)
    x2, acc = _pass1_call(ints, ga, gb, pred_vecs, pca_W1,
                          pca_b1.reshape(1, -1), clf_W1,
                          clf_b1.reshape(1, -1), patch)
    outs, outp, outo = _pass2_call(x2, acc, clf_W1, clf_b1.reshape(1, -1),
                                   bn1_gamma.reshape(1, -1),
                                   bn1_beta.reshape(1, -1))
    pp = _sc_scatter(outs, outo, s_idx, o_idx, zrow)
    obj_out = _final_call(ints, pp, cc, pca_W2, pca_b2.reshape(1, -1),
                          clf_W2, clf_b2.reshape(1, -1),
                          bn2_gamma.reshape(1, -1), bn2_beta.reshape(1, -1))
    return (obj_out, outp)
